# Initial kernel scaffold; baseline (speedup 1.0000x reference)
import functools

import jax
import jax.numpy as jnp
from jax import lax
from jax.experimental import pallas as pl
from jax.experimental.pallas import tpu as pltpu
from jax.experimental.pallas import tpu_sc as plsc

N = 10000
E = 320000
U = 128

NC = 2
NS = 16
NW = NC * NS
PER_W = E // NW
T = 80
NT = PER_W // T
RPS = N // NS
ZR = 125

_mesh = functools.partial(
    plsc.VectorSubcoreMesh, core_axis_name="c", subcore_axis_name="s")


def _ew_loop(nrows, fn):
    def body(r, _):
        for cc in range(U // 16):
            fn(r, cc * 16)
        return 0
    lax.fori_loop(0, nrows, body, 0)


def _sc_gather_sum(Ps, Pd, src, dst):
    @functools.partial(
        pl.kernel,
        mesh=_mesh(),
        out_type=jax.ShapeDtypeStruct((E, U), jnp.float32),
        scratch_types=[
            pltpu.VMEM((T,), jnp.int32),
            pltpu.VMEM((T,), jnp.int32),
            pltpu.VMEM((T, U), jnp.float32),
            pltpu.VMEM((T, U), jnp.float32),
            pltpu.SemaphoreType.DMA,
        ],
    )
    def k(ps_h, pd_h, src_h, dst_h, s_out, idx_s, idx_d, g1, g2, sem):
        wid = lax.axis_index("s") * NC + lax.axis_index("c")
        base0 = wid * PER_W

        def tile(t, _):
            base = base0 + t * T
            pltpu.sync_copy(src_h.at[pl.ds(base, T)], idx_s)
            pltpu.sync_copy(dst_h.at[pl.ds(base, T)], idx_d)
            pltpu.async_copy(ps_h.at[idx_s], g1, sem).wait()
            pltpu.async_copy(pd_h.at[idx_d], g2, sem).wait()

            def add(r, c):
                g1[r, pl.ds(c, 16)] = g1[r, pl.ds(c, 16)] + g2[r, pl.ds(c, 16)]
            _ew_loop(T, add)
            pltpu.sync_copy(g1, s_out.at[pl.ds(base, T)])
            return 0

        lax.fori_loop(0, NT, tile, 0)

    return k(Ps, Pd, src, dst)


def _sc_msg_agg(Pm, sig, src, dst):
    @functools.partial(
        pl.kernel,
        mesh=_mesh(),
        out_type=jax.ShapeDtypeStruct((NC, N, U), jnp.float32),
        scratch_types=[
            pltpu.VMEM((T,), jnp.int32),
            pltpu.VMEM((T,), jnp.int32),
            pltpu.VMEM((T, U), jnp.float32),
            pltpu.VMEM((T, U), jnp.float32),
            pltpu.VMEM((ZR, U), jnp.float32),
            pltpu.VMEM_SHARED((N, U), jnp.float32),
            pltpu.SemaphoreType.DMA,
        ],
    )
    def k(pm_h, sig_h, src_h, dst_h, out_h, idx_s, idx_d, g, sg, stage, agg, sem):
        cid = lax.axis_index("c")
        sid = lax.axis_index("s")
        wid = sid * NC + cid
        base0 = wid * PER_W

        def zero(r, c):
            stage[r, pl.ds(c, 16)] = jnp.zeros((16,), jnp.float32)
        _ew_loop(ZR, zero)
        for q in range(RPS // ZR):
            pltpu.sync_copy(stage, agg.at[pl.ds(sid * RPS + q * ZR, ZR)])
        plsc.subcore_barrier()

        def tile(t, _):
            base = base0 + t * T
            pltpu.sync_copy(src_h.at[pl.ds(base, T)], idx_s)
            pltpu.sync_copy(dst_h.at[pl.ds(base, T)], idx_d)
            pltpu.async_copy(pm_h.at[idx_s], g, sem).wait()
            pltpu.sync_copy(sig_h.at[pl.ds(base, T)], sg)

            def mul(r, c):
                g[r, pl.ds(c, 16)] = g[r, pl.ds(c, 16)] * sg[r, pl.ds(c, 16)]
            _ew_loop(T, mul)
            pltpu.sync_copy(g, agg.at[idx_d], add=True)
            return 0

        lax.fori_loop(0, NT, tile, 0)
        plsc.subcore_barrier()

        for q in range(RPS // ZR):
            row = sid * RPS + q * ZR
            pltpu.sync_copy(agg.at[pl.ds(row, ZR)], stage)
            pltpu.sync_copy(stage, out_h.at[cid, pl.ds(row, ZR)])

    return k(Pm, sig, src, dst)


EB = 2000


def _tc_node_init(pos, W_pos, b_pos, Wm, bm, Ws, Wd, Wr, br):
    def body(pos_r, wp, bp, wm, bm_r, ws, wd, wr, br_r,
             x_o, pm_o, ps_o, pd_o, rr_o):
        x = jnp.dot(pos_r[...], wp[...],
                    preferred_element_type=jnp.float32) + bp[...]
        x_o[...] = x
        pm_o[...] = jnp.dot(x, wm[...], preferred_element_type=jnp.float32) + bm_r[...]
        ps_o[...] = jnp.dot(x, ws[...], preferred_element_type=jnp.float32)
        pd_o[...] = jnp.dot(x, wd[...], preferred_element_type=jnp.float32)
        rr_o[...] = jnp.dot(x, wr[...], preferred_element_type=jnp.float32) + br_r[...]

    out = jax.ShapeDtypeStruct((N, U), jnp.float32)
    return pl.pallas_call(
        body,
        out_shape=(out,) * 5,
    )(pos, W_pos, b_pos.reshape(1, U), Wm, bm.reshape(1, U), Ws, Wd, Wr,
      br.reshape(1, U))


def _tc_node_update(x, R, agg0, agg1, ws, bs):
    n_proj = len(ws)

    def body(x_r, r_r, a0, a1, *rest):
        w_refs = rest[:n_proj]
        b_refs = rest[n_proj:2 * n_proj]
        outs = rest[2 * n_proj:]
        x1 = x_r[...] + jax.nn.relu(r_r[...] + a0[...] + a1[...])
        outs[0][...] = x1
        for wo, bo, oo in zip(w_refs, b_refs, outs[1:]):
            oo[...] = jnp.dot(x1, wo[...],
                              preferred_element_type=jnp.float32) + bo[...]

    out = jax.ShapeDtypeStruct((N, U), jnp.float32)
    return pl.pallas_call(
        body,
        out_shape=(out,) * (n_proj + 1),
    )(x, R, agg0, agg1, *ws, *[b.reshape(1, U) for b in bs])


def _tc_edge_init(edge_attr, W_attr, b_attr):
    def body(ea, wa, ba, e_o, sig_o):
        e = jnp.dot(ea[...], wa[...],
                    preferred_element_type=jnp.float32) + ba[...]
        e_o[...] = e
        sig_o[...] = jax.nn.sigmoid(e)

    out = jax.ShapeDtypeStruct((E, U), jnp.float32)
    grid = (E // EB,)
    return pl.pallas_call(
        body,
        grid=grid,
        in_specs=[
            pl.BlockSpec((EB, 5), lambda i: (i, 0)),
            pl.BlockSpec((5, U), lambda i: (0, 0)),
            pl.BlockSpec((1, U), lambda i: (0, 0)),
        ],
        out_specs=(pl.BlockSpec((EB, U), lambda i: (i, 0)),) * 2,
        out_shape=(out, out),
    )(edge_attr, W_attr, b_attr.reshape(1, U))


def _tc_edge_linear(e, S, We, be, want_sig):
    def body(e_r, s_r, w, b, *outs):
        ev = e_r[...]
        enew = ev + jax.nn.relu(
            s_r[...] + jnp.dot(ev, w[...], preferred_element_type=jnp.float32)
            + b[...])
        outs[0][...] = enew
        if want_sig:
            outs[1][...] = jax.nn.sigmoid(enew)

    out = jax.ShapeDtypeStruct((E, U), jnp.float32)
    n_out = 2 if want_sig else 1
    grid = (E // EB,)
    res = pl.pallas_call(
        body,
        grid=grid,
        in_specs=[
            pl.BlockSpec((EB, U), lambda i: (i, 0)),
            pl.BlockSpec((EB, U), lambda i: (i, 0)),
            pl.BlockSpec((U, U), lambda i: (0, 0)),
            pl.BlockSpec((1, U), lambda i: (0, 0)),
        ],
        out_specs=(pl.BlockSpec((EB, U), lambda i: (i, 0)),) * n_out,
        out_shape=(out,) * n_out,
    )(e, S, We, be.reshape(1, U))
    return res if want_sig else (res[0], None)


def _tc_final(e, S, ea0, We, be, W1a, w1b, b1, a, W2, b2):
    H = W1a.shape[1]

    def body(e_r, s_r, ea_r, w, b, w1, w1b_r, b1_r, a_r, w2, b2_r, out_r):
        ev = e_r[...]
        ef = ev + jax.nn.relu(
            s_r[...] + jnp.dot(ev, w[...], preferred_element_type=jnp.float32)
            + b[...])
        h = (jnp.dot(ef, w1[...], preferred_element_type=jnp.float32)
             + ea_r[...] * w1b_r[...] + b1_r[...])
        h = jnp.where(h >= 0, h, a_r[...] * h)
        out_r[...] = jnp.dot(h, w2[...],
                             preferred_element_type=jnp.float32) + b2_r[...]

    grid = (E // EB,)
    return pl.pallas_call(
        body,
        grid=grid,
        in_specs=[
            pl.BlockSpec((EB, U), lambda i: (i, 0)),
            pl.BlockSpec((EB, U), lambda i: (i, 0)),
            pl.BlockSpec((EB, 1), lambda i: (i, 0)),
            pl.BlockSpec((U, U), lambda i: (0, 0)),
            pl.BlockSpec((1, U), lambda i: (0, 0)),
            pl.BlockSpec((U, H), lambda i: (0, 0)),
            pl.BlockSpec((1, H), lambda i: (0, 0)),
            pl.BlockSpec((1, H), lambda i: (0, 0)),
            pl.BlockSpec((1, 1), lambda i: (0, 0)),
            pl.BlockSpec((H, 1), lambda i: (0, 0)),
            pl.BlockSpec((1, 1), lambda i: (0, 0)),
        ],
        out_specs=pl.BlockSpec((EB, 1), lambda i: (i, 0)),
        out_shape=jax.ShapeDtypeStruct((E, 1), jnp.float32),
    )(e, S, ea0, We, be.reshape(1, U), W1a, w1b, b1.reshape(1, H),
      a.reshape(1, 1), W2, b2.reshape(1, 1))


def kernel(pos, edge_attr, edge_index, W_pos, b_pos, W_attr, b_attr,
           nc_W_root, nc_b_root, nc_W_msg, nc_b_msg, el_W_src, el_W_dst,
           el_W_e, el_b, mlp_W1, mlp_b1, prelu_a, mlp_W2, mlp_b2):
    src = edge_index[0]
    dst = edge_index[1]
    zb = jnp.zeros((U,), jnp.float32)

    x0, Pm0, Ps0, Pd0, R0 = _tc_node_init(
        pos, W_pos, b_pos, nc_W_msg[0], nc_b_msg[0], el_W_src[0], el_W_dst[0],
        nc_W_root[0], nc_b_root[0])
    e0, sig0 = _tc_edge_init(edge_attr, W_attr, b_attr)

    S0 = _sc_gather_sum(Ps0, Pd0, src, dst)
    agg0 = _sc_msg_agg(Pm0, sig0, src, dst)
    e1, sig1 = _tc_edge_linear(e0, S0, el_W_e[0], el_b[0], want_sig=True)
    x1, Pm1, Ps1, Pd1, R1 = _tc_node_update(
        x0, R0, agg0[0], agg0[1],
        [nc_W_msg[1], el_W_src[1], el_W_dst[1], nc_W_root[1]],
        [nc_b_msg[1], zb, zb, nc_b_root[1]])

    S1 = _sc_gather_sum(Ps1, Pd1, src, dst)
    agg1 = _sc_msg_agg(Pm1, sig1, src, dst)
    e2, _ = _tc_edge_linear(e1, S1, el_W_e[1], el_b[1], want_sig=False)
    x2, Ps2, Pd2 = _tc_node_update(
        x1, R1, agg1[0], agg1[1],
        [el_W_src[2], el_W_dst[2]], [zb, zb])

    S2 = _sc_gather_sum(Ps2, Pd2, src, dst)
    ea0 = edge_attr[:, :1]
    W1a = mlp_W1[:U]
    w1b = mlp_W1[U:U + 1]
    logits = _tc_final(e2, S2, ea0, el_W_e[2], el_b[2], W1a, w1b, mlp_b1,
                       prelu_a, mlp_W2, mlp_b2)
    return logits


# speedup vs baseline: 1.5677x; 1.5677x over previous
import functools

import jax
import jax.numpy as jnp
from jax import lax
from jax.experimental import pallas as pl
from jax.experimental.pallas import tpu as pltpu
from jax.experimental.pallas import tpu_sc as plsc

N = 10000
E = 320000
U = 128

NC = 2
NS = 16
NW = NC * NS
PER_W = E // NW
T = 80
NT = PER_W // T
ZR = 80
NCH = N // ZR
CPS = -(-NCH // NS)

_mesh = functools.partial(
    plsc.VectorSubcoreMesh, core_axis_name="c", subcore_axis_name="s")


def _ew_loop(nrows, fn):
    def body(r, _):
        for cc in range(U // 16):
            fn(r, cc * 16)
        return 0
    lax.fori_loop(0, nrows, body, 0)


def _sc_gather_sum(Ps, Pd, src, dst):
    @functools.partial(
        pl.kernel,
        mesh=_mesh(),
        out_type=jax.ShapeDtypeStruct((E, U), jnp.float32),
        scratch_types=[
            pltpu.VMEM((T,), jnp.int32),
            pltpu.VMEM((T,), jnp.int32),
            pltpu.VMEM((T, U), jnp.float32),
            pltpu.VMEM((T, U), jnp.float32),
            pltpu.SemaphoreType.DMA,
        ],
    )
    def k(ps_h, pd_h, src_h, dst_h, s_out, idx_s, idx_d, g1, g2, sem):
        wid = lax.axis_index("s") * NC + lax.axis_index("c")
        base0 = wid * PER_W

        def tile(t, _):
            base = base0 + t * T
            pltpu.sync_copy(src_h.at[pl.ds(base, T)], idx_s)
            pltpu.sync_copy(dst_h.at[pl.ds(base, T)], idx_d)
            pltpu.async_copy(ps_h.at[idx_s], g1, sem).wait()
            pltpu.async_copy(pd_h.at[idx_d], g2, sem).wait()

            def add(r, c):
                g1[r, pl.ds(c, 16)] = g1[r, pl.ds(c, 16)] + g2[r, pl.ds(c, 16)]
            _ew_loop(T, add)
            pltpu.sync_copy(g1, s_out.at[pl.ds(base, T)])
            return 0

        lax.fori_loop(0, NT, tile, 0)

    return k(Ps, Pd, src, dst)


def _sc_msg_agg(Pm, sig, src, dst):
    @functools.partial(
        pl.kernel,
        mesh=_mesh(),
        out_type=jax.ShapeDtypeStruct((NC, N, U), jnp.float32),
        scratch_types=[
            pltpu.VMEM((T,), jnp.int32),
            pltpu.VMEM((T,), jnp.int32),
            pltpu.VMEM((T, U), jnp.float32),
            pltpu.VMEM((T, U), jnp.float32),
            pltpu.VMEM((ZR, U), jnp.float32),
            pltpu.VMEM_SHARED((N, U), jnp.float32),
            pltpu.SemaphoreType.DMA,
        ],
    )
    def k(pm_h, sig_h, src_h, dst_h, out_h, idx_s, idx_d, g, sg, stage, agg, sem):
        cid = lax.axis_index("c")
        sid = lax.axis_index("s")
        wid = sid * NC + cid
        base0 = wid * PER_W

        def zero(r, c):
            stage[r, pl.ds(c, 16)] = jnp.zeros((16,), jnp.float32)
        _ew_loop(ZR, zero)
        for q in range(CPS):
            ch = sid + q * NS

            @pl.when(ch < NCH)
            def _():
                pltpu.sync_copy(stage, agg.at[pl.ds(ch * ZR, ZR)])
        plsc.subcore_barrier()

        def tile(t, _):
            base = base0 + t * T
            pltpu.sync_copy(src_h.at[pl.ds(base, T)], idx_s)
            pltpu.sync_copy(dst_h.at[pl.ds(base, T)], idx_d)
            pltpu.async_copy(pm_h.at[idx_s], g, sem).wait()
            pltpu.sync_copy(sig_h.at[pl.ds(base, T)], sg)

            def mul(r, c):
                g[r, pl.ds(c, 16)] = g[r, pl.ds(c, 16)] * sg[r, pl.ds(c, 16)]
            _ew_loop(T, mul)
            pltpu.sync_copy(g, agg.at[idx_d], add=True)
            return 0

        lax.fori_loop(0, NT, tile, 0)
        plsc.subcore_barrier()

        for q in range(CPS):
            ch = sid + q * NS

            @pl.when(ch < NCH)
            def _():
                pltpu.sync_copy(agg.at[pl.ds(ch * ZR, ZR)], stage)
                pltpu.sync_copy(stage, out_h.at[cid, pl.ds(ch * ZR, ZR)])

    return k(Pm, sig, src, dst)


EB = 2000


def _tc_node_init(pos, W_pos, b_pos, Wm, bm, Ws, Wd, Wr, br):
    def body(pos_r, wp, bp, wm, bm_r, ws, wd, wr, br_r,
             x_o, pm_o, ps_o, pd_o, rr_o):
        x = jnp.dot(pos_r[...], wp[...],
                    preferred_element_type=jnp.float32) + bp[...]
        x_o[...] = x
        pm_o[...] = jnp.dot(x, wm[...], preferred_element_type=jnp.float32) + bm_r[...]
        ps_o[...] = jnp.dot(x, ws[...], preferred_element_type=jnp.float32)
        pd_o[...] = jnp.dot(x, wd[...], preferred_element_type=jnp.float32)
        rr_o[...] = jnp.dot(x, wr[...], preferred_element_type=jnp.float32) + br_r[...]

    out = jax.ShapeDtypeStruct((N, U), jnp.float32)
    return pl.pallas_call(
        body,
        out_shape=(out,) * 5,
    )(pos, W_pos, b_pos.reshape(1, U), Wm, bm.reshape(1, U), Ws, Wd, Wr,
      br.reshape(1, U))


def _tc_node_update(x, R, agg0, agg1, ws, bs):
    n_proj = len(ws)

    def body(x_r, r_r, a0, a1, *rest):
        w_refs = rest[:n_proj]
        b_refs = rest[n_proj:2 * n_proj]
        outs = rest[2 * n_proj:]
        x1 = x_r[...] + jax.nn.relu(r_r[...] + a0[...] + a1[...])
        outs[0][...] = x1
        for wo, bo, oo in zip(w_refs, b_refs, outs[1:]):
            oo[...] = jnp.dot(x1, wo[...],
                              preferred_element_type=jnp.float32) + bo[...]

    out = jax.ShapeDtypeStruct((N, U), jnp.float32)
    return pl.pallas_call(
        body,
        out_shape=(out,) * (n_proj + 1),
    )(x, R, agg0, agg1, *ws, *[b.reshape(1, U) for b in bs])


def _tc_edge_init(edge_attr, W_attr, b_attr):
    def body(ea, wa, ba, e_o, sig_o):
        e = jnp.dot(ea[...], wa[...],
                    preferred_element_type=jnp.float32) + ba[...]
        e_o[...] = e
        sig_o[...] = jax.nn.sigmoid(e)

    out = jax.ShapeDtypeStruct((E, U), jnp.float32)
    grid = (E // EB,)
    return pl.pallas_call(
        body,
        grid=grid,
        in_specs=[
            pl.BlockSpec((EB, 5), lambda i: (i, 0)),
            pl.BlockSpec((5, U), lambda i: (0, 0)),
            pl.BlockSpec((1, U), lambda i: (0, 0)),
        ],
        out_specs=(pl.BlockSpec((EB, U), lambda i: (i, 0)),) * 2,
        out_shape=(out, out),
    )(edge_attr, W_attr, b_attr.reshape(1, U))


def _tc_edge_linear(e, S, We, be, want_sig):
    def body(e_r, s_r, w, b, *outs):
        ev = e_r[...]
        enew = ev + jax.nn.relu(
            s_r[...] + jnp.dot(ev, w[...], preferred_element_type=jnp.float32)
            + b[...])
        outs[0][...] = enew
        if want_sig:
            outs[1][...] = jax.nn.sigmoid(enew)

    out = jax.ShapeDtypeStruct((E, U), jnp.float32)
    n_out = 2 if want_sig else 1
    grid = (E // EB,)
    res = pl.pallas_call(
        body,
        grid=grid,
        in_specs=[
            pl.BlockSpec((EB, U), lambda i: (i, 0)),
            pl.BlockSpec((EB, U), lambda i: (i, 0)),
            pl.BlockSpec((U, U), lambda i: (0, 0)),
            pl.BlockSpec((1, U), lambda i: (0, 0)),
        ],
        out_specs=(pl.BlockSpec((EB, U), lambda i: (i, 0)),) * n_out,
        out_shape=(out,) * n_out,
    )(e, S, We, be.reshape(1, U))
    return res if want_sig else (res[0], None)


def _tc_final(e, S, ea0, We, be, W1a, w1b, b1, a, W2, b2):
    H = W1a.shape[1]

    def body(e_r, s_r, ea_r, w, b, w1, w1b_r, b1_r, a_r, w2, b2_r, out_r):
        ev = e_r[...]
        ef = ev + jax.nn.relu(
            s_r[...] + jnp.dot(ev, w[...], preferred_element_type=jnp.float32)
            + b[...])
        h = (jnp.dot(ef, w1[...], preferred_element_type=jnp.float32)
             + ea_r[...] * w1b_r[...] + b1_r[...])
        h = jnp.where(h >= 0, h, a_r[...] * h)
        out_r[...] = jnp.dot(h, w2[...],
                             preferred_element_type=jnp.float32) + b2_r[...]

    grid = (E // EB,)
    return pl.pallas_call(
        body,
        grid=grid,
        in_specs=[
            pl.BlockSpec((EB, U), lambda i: (i, 0)),
            pl.BlockSpec((EB, U), lambda i: (i, 0)),
            pl.BlockSpec((EB, 1), lambda i: (i, 0)),
            pl.BlockSpec((U, U), lambda i: (0, 0)),
            pl.BlockSpec((1, U), lambda i: (0, 0)),
            pl.BlockSpec((U, H), lambda i: (0, 0)),
            pl.BlockSpec((1, H), lambda i: (0, 0)),
            pl.BlockSpec((1, H), lambda i: (0, 0)),
            pl.BlockSpec((1, 1), lambda i: (0, 0)),
            pl.BlockSpec((H, 1), lambda i: (0, 0)),
            pl.BlockSpec((1, 1), lambda i: (0, 0)),
        ],
        out_specs=pl.BlockSpec((EB, 1), lambda i: (i, 0)),
        out_shape=jax.ShapeDtypeStruct((E, 1), jnp.float32),
    )(e, S, ea0, We, be.reshape(1, U), W1a, w1b, b1.reshape(1, H),
      a.reshape(1, 1), W2, b2.reshape(1, 1))


def kernel(pos, edge_attr, edge_index, W_pos, b_pos, W_attr, b_attr,
           nc_W_root, nc_b_root, nc_W_msg, nc_b_msg, el_W_src, el_W_dst,
           el_W_e, el_b, mlp_W1, mlp_b1, prelu_a, mlp_W2, mlp_b2):
    src = edge_index[0]
    dst = edge_index[1]
    zb = jnp.zeros((U,), jnp.float32)

    x0, Pm0, Ps0, Pd0, R0 = _tc_node_init(
        pos, W_pos, b_pos, nc_W_msg[0], nc_b_msg[0], el_W_src[0], el_W_dst[0],
        nc_W_root[0], nc_b_root[0])
    e0, sig0 = _tc_edge_init(edge_attr, W_attr, b_attr)

    S0 = _sc_gather_sum(Ps0, Pd0, src, dst)
    agg0 = _sc_msg_agg(Pm0, sig0, src, dst)
    e1, sig1 = _tc_edge_linear(e0, S0, el_W_e[0], el_b[0], want_sig=True)
    x1, Pm1, Ps1, Pd1, R1 = _tc_node_update(
        x0, R0, agg0[0], agg0[1],
        [nc_W_msg[1], el_W_src[1], el_W_dst[1], nc_W_root[1]],
        [nc_b_msg[1], zb, zb, nc_b_root[1]])

    S1 = _sc_gather_sum(Ps1, Pd1, src, dst)
    agg1 = _sc_msg_agg(Pm1, sig1, src, dst)
    e2, _ = _tc_edge_linear(e1, S1, el_W_e[1], el_b[1], want_sig=False)
    x2, Ps2, Pd2 = _tc_node_update(
        x1, R1, agg1[0], agg1[1],
        [el_W_src[2], el_W_dst[2]], [zb, zb])

    S2 = _sc_gather_sum(Ps2, Pd2, src, dst)
    ea0 = edge_attr[:, :1]
    W1a = mlp_W1[:U]
    w1b = mlp_W1[U:U + 1]
    logits = _tc_final(e2, S2, ea0, el_W_e[2], el_b[2], W1a, w1b, mlp_b1,
                       prelu_a, mlp_W2, mlp_b2)
    return logits


# speedup vs baseline: 2.4208x; 1.5442x over previous
import functools

import jax
import jax.numpy as jnp
from jax import lax
from jax.experimental import pallas as pl
from jax.experimental.pallas import tpu as pltpu
from jax.experimental.pallas import tpu_sc as plsc

N = 10000
E = 320000
U = 128

NC = 2
NS = 16
NW = NC * NS
PER_W = E // NW
T = 80
NT = PER_W // T
NP = (NT - 1) // 2
ZR = 80
NCH = N // ZR
CPS = -(-NCH // NS)

_mesh = functools.partial(
    plsc.VectorSubcoreMesh, core_axis_name="c", subcore_axis_name="s")


def _sc_msg_agg(Pm, sig, src, dst):
    @functools.partial(
        pl.kernel,
        mesh=_mesh(),
        out_type=jax.ShapeDtypeStruct((NC, N, U), jnp.float32),
        scratch_types=[
            pltpu.VMEM((T,), jnp.int32), pltpu.VMEM((T,), jnp.int32),
            pltpu.VMEM((T,), jnp.int32), pltpu.VMEM((T,), jnp.int32),
            pltpu.VMEM((T, U), jnp.float32), pltpu.VMEM((T, U), jnp.float32),
            pltpu.VMEM((T, U), jnp.float32), pltpu.VMEM((T, U), jnp.float32),
            pltpu.VMEM_SHARED((N, U), jnp.float32),
            pltpu.SemaphoreType.DMA, pltpu.SemaphoreType.DMA,
        ],
    )
    def k(pm_h, sig_h, src_h, dst_h, agg_out,
          ixs0, ixs1, ixd0, ixd1, g0, g1, sg0, sg1, agg, sem0, sem1):
        cid = lax.axis_index("c")
        sid = lax.axis_index("s")
        wid = sid * NC + cid
        base0 = wid * PER_W
        bufs = ((ixs0, ixd0, g0, sg0, sem0), (ixs1, ixd1, g1, sg1, sem1))

        def zrow(r, _):
            for cc in range(U // 16):
                g0[r, pl.ds(cc * 16, 16)] = jnp.zeros((16,), jnp.float32)
            return 0
        lax.fori_loop(0, ZR, zrow, 0)
        for q in range(CPS):
            ch = sid + q * NS

            @pl.when(ch < NCH)
            def _():
                pltpu.sync_copy(g0, agg.at[pl.ds(ch * ZR, ZR)])
        plsc.subcore_barrier()

        def start(t, b):
            ixs, ixd, g, sg, sem = bufs[b]
            base = base0 + t * T
            pltpu.sync_copy(src_h.at[pl.ds(base, T)], ixs)
            pltpu.sync_copy(dst_h.at[pl.ds(base, T)], ixd)
            pltpu.async_copy(pm_h.at[ixs], g, sem)
            pltpu.async_copy(sig_h.at[pl.ds(base, T)], sg, sem)

        def finish(t, b):
            ixs, ixd, g, sg, sem = bufs[b]
            base = base0 + t * T
            pltpu.make_async_copy(pm_h.at[ixs], g, sem).wait()
            pltpu.make_async_copy(sig_h.at[pl.ds(base, T)], sg, sem).wait()

            def row(r, _):
                for cc in range(U // 16):
                    c = cc * 16
                    g[r, pl.ds(c, 16)] = (g[r, pl.ds(c, 16)]
                                          * sg[r, pl.ds(c, 16)])
                return 0
            lax.fori_loop(0, T, row, 0)
            pltpu.sync_copy(g, agg.at[ixd], add=True)

        start(0, 0)

        def pair(i, _):
            g = 2 * i
            start(g + 1, 1)
            finish(g, 0)
            start(g + 2, 0)
            finish(g + 1, 1)
            return 0

        lax.fori_loop(0, NP, pair, 0)
        finish(NT - 1, 0)
        plsc.subcore_barrier()

        for q in range(CPS):
            ch = sid + q * NS

            @pl.when(ch < NCH)
            def _():
                pltpu.sync_copy(agg.at[pl.ds(ch * ZR, ZR)], g0)
                pltpu.sync_copy(g0, agg_out.at[cid, pl.ds(ch * ZR, ZR)])

    return k(Pm, sig, src, dst)


def _sc_gather_sum(Ps, Pd, src, dst):
    @functools.partial(
        pl.kernel,
        mesh=_mesh(),
        out_type=jax.ShapeDtypeStruct((E, U), jnp.float32),
        scratch_types=[
            pltpu.VMEM((T,), jnp.int32), pltpu.VMEM((T,), jnp.int32),
            pltpu.VMEM((T,), jnp.int32), pltpu.VMEM((T,), jnp.int32),
            pltpu.VMEM((T, U), jnp.float32), pltpu.VMEM((T, U), jnp.float32),
            pltpu.VMEM((T, U), jnp.float32), pltpu.VMEM((T, U), jnp.float32),
            pltpu.SemaphoreType.DMA, pltpu.SemaphoreType.DMA,
        ],
    )
    def k(ps_h, pd_h, src_h, dst_h, s_out,
          ixs0, ixs1, ixd0, ixd1, g10, g11, g20, g21, sem0, sem1):
        wid = lax.axis_index("s") * NC + lax.axis_index("c")
        base0 = wid * PER_W
        bufs = ((ixs0, ixd0, g10, g20, sem0), (ixs1, ixd1, g11, g21, sem1))

        def start(t, b):
            ixs, ixd, g1, g2, sem = bufs[b]
            base = base0 + t * T
            pltpu.sync_copy(src_h.at[pl.ds(base, T)], ixs)
            pltpu.sync_copy(dst_h.at[pl.ds(base, T)], ixd)
            pltpu.async_copy(ps_h.at[ixs], g1, sem)
            pltpu.async_copy(pd_h.at[ixd], g2, sem)

        def finish(t, b):
            ixs, ixd, g1, g2, sem = bufs[b]
            base = base0 + t * T
            pltpu.make_async_copy(ps_h.at[ixs], g1, sem).wait()
            pltpu.make_async_copy(pd_h.at[ixd], g2, sem).wait()

            def row(r, _):
                for cc in range(U // 16):
                    c = cc * 16
                    g1[r, pl.ds(c, 16)] = (g1[r, pl.ds(c, 16)]
                                           + g2[r, pl.ds(c, 16)])
                return 0
            lax.fori_loop(0, T, row, 0)
            pltpu.sync_copy(g1, s_out.at[pl.ds(base, T)])

        start(0, 0)

        def pair(i, _):
            g = 2 * i
            start(g + 1, 1)
            finish(g, 0)
            start(g + 2, 0)
            finish(g + 1, 1)
            return 0

        lax.fori_loop(0, NP, pair, 0)
        finish(NT - 1, 0)

    return k(Ps, Pd, src, dst)


EB = 2000


def _tc_node_init(pos, W_pos, b_pos, ws, bs):
    n_proj = len(ws)

    def body(pos_r, wp, bp, *rest):
        w_refs = rest[:n_proj]
        b_refs = rest[n_proj:2 * n_proj]
        outs = rest[2 * n_proj:]
        x = jnp.dot(pos_r[...], wp[...],
                    preferred_element_type=jnp.float32) + bp[...]
        outs[0][...] = x
        for wo, bo, oo in zip(w_refs, b_refs, outs[1:]):
            oo[...] = jnp.dot(x, wo[...],
                              preferred_element_type=jnp.float32) + bo[...]

    outs = (jax.ShapeDtypeStruct((N, U), jnp.float32),) + tuple(
        jax.ShapeDtypeStruct((N, w.shape[1]), jnp.float32) for w in ws)
    return pl.pallas_call(body, out_shape=outs)(
        pos, W_pos, b_pos.reshape(1, U), *ws,
        *[b.reshape(1, -1) for b in bs])


def _tc_node_update(x, R, agg0, agg1, ws, bs):
    n_proj = len(ws)

    def body(x_r, r_r, a0, a1, *rest):
        w_refs = rest[:n_proj]
        b_refs = rest[n_proj:2 * n_proj]
        outs = rest[2 * n_proj:]
        x1 = x_r[...] + jax.nn.relu(r_r[...] + a0[...] + a1[...])
        outs[0][...] = x1
        for wo, bo, oo in zip(w_refs, b_refs, outs[1:]):
            oo[...] = jnp.dot(x1, wo[...],
                              preferred_element_type=jnp.float32) + bo[...]

    outs = (jax.ShapeDtypeStruct((N, U), jnp.float32),) + tuple(
        jax.ShapeDtypeStruct((N, w.shape[1]), jnp.float32) for w in ws)
    return pl.pallas_call(body, out_shape=outs)(
        x, R, agg0, agg1, *ws, *[b.reshape(1, -1) for b in bs])


def _tc_edge_init(edge_attr, W_attr, b_attr):
    def body(ea, wa, ba, e_o, sig_o):
        e = jnp.dot(ea[...], wa[...],
                    preferred_element_type=jnp.float32) + ba[...]
        e_o[...] = e
        sig_o[...] = jax.nn.sigmoid(e)

    out = jax.ShapeDtypeStruct((E, U), jnp.float32)
    grid = (E // EB,)
    return pl.pallas_call(
        body,
        grid=grid,
        in_specs=[
            pl.BlockSpec((EB, 5), lambda i: (i, 0)),
            pl.BlockSpec((5, U), lambda i: (0, 0)),
            pl.BlockSpec((1, U), lambda i: (0, 0)),
        ],
        out_specs=(pl.BlockSpec((EB, U), lambda i: (i, 0)),) * 2,
        out_shape=(out, out),
    )(edge_attr, W_attr, b_attr.reshape(1, U))


def _tc_edge_linear(e, S, We, be, want_sig):
    def body(e_r, s_r, w, b, *outs):
        ev = e_r[...]
        enew = ev + jax.nn.relu(
            s_r[...] + jnp.dot(ev, w[...], preferred_element_type=jnp.float32)
            + b[...])
        outs[0][...] = enew
        if want_sig:
            outs[1][...] = jax.nn.sigmoid(enew)

    out = jax.ShapeDtypeStruct((E, U), jnp.float32)
    n_out = 2 if want_sig else 1
    grid = (E // EB,)
    res = pl.pallas_call(
        body,
        grid=grid,
        in_specs=[
            pl.BlockSpec((EB, U), lambda i: (i, 0)),
            pl.BlockSpec((EB, U), lambda i: (i, 0)),
            pl.BlockSpec((U, U), lambda i: (0, 0)),
            pl.BlockSpec((1, U), lambda i: (0, 0)),
        ],
        out_specs=(pl.BlockSpec((EB, U), lambda i: (i, 0)),) * n_out,
        out_shape=(out,) * n_out,
    )(e, S, We, be.reshape(1, U))
    return res if want_sig else (res[0], None)


def _tc_final(e, S, ea0, We, be, W1a, w1b, b1, a, W2, b2):
    H = W1a.shape[1]

    def body(e_r, s_r, ea_r, w, b, w1, w1b_r, b1_r, a_r, w2, b2_r, out_r):
        ev = e_r[...]
        ef = ev + jax.nn.relu(
            s_r[...] + jnp.dot(ev, w[...], preferred_element_type=jnp.float32)
            + b[...])
        h = (jnp.dot(ef, w1[...], preferred_element_type=jnp.float32)
             + ea_r[...] * w1b_r[...] + b1_r[...])
        h = jnp.where(h >= 0, h, a_r[...] * h)
        out_r[...] = jnp.dot(h, w2[...],
                             preferred_element_type=jnp.float32) + b2_r[...]

    grid = (E // EB,)
    return pl.pallas_call(
        body,
        grid=grid,
        in_specs=[
            pl.BlockSpec((EB, U), lambda i: (i, 0)),
            pl.BlockSpec((EB, U), lambda i: (i, 0)),
            pl.BlockSpec((EB, 1), lambda i: (i, 0)),
            pl.BlockSpec((U, U), lambda i: (0, 0)),
            pl.BlockSpec((1, U), lambda i: (0, 0)),
            pl.BlockSpec((U, H), lambda i: (0, 0)),
            pl.BlockSpec((1, H), lambda i: (0, 0)),
            pl.BlockSpec((1, H), lambda i: (0, 0)),
            pl.BlockSpec((1, 1), lambda i: (0, 0)),
            pl.BlockSpec((H, 1), lambda i: (0, 0)),
            pl.BlockSpec((1, 1), lambda i: (0, 0)),
        ],
        out_specs=pl.BlockSpec((EB, 1), lambda i: (i, 0)),
        out_shape=jax.ShapeDtypeStruct((E, 1), jnp.float32),
    )(e, S, ea0, We, be.reshape(1, U), W1a, w1b, b1.reshape(1, H),
      a.reshape(1, 1), W2, b2.reshape(1, 1))


def kernel(pos, edge_attr, edge_index, W_pos, b_pos, W_attr, b_attr,
           nc_W_root, nc_b_root, nc_W_msg, nc_b_msg, el_W_src, el_W_dst,
           el_W_e, el_b, mlp_W1, mlp_b1, prelu_a, mlp_W2, mlp_b2):
    src = edge_index[0]
    dst = edge_index[1]
    zb = jnp.zeros((U,), jnp.float32)

    x0, Ps0, Pm0, Pd0, R0 = _tc_node_init(
        pos, W_pos, b_pos,
        [el_W_src[0], nc_W_msg[0], el_W_dst[0], nc_W_root[0]],
        [zb, nc_b_msg[0], zb, nc_b_root[0]])
    e0, sig0 = _tc_edge_init(edge_attr, W_attr, b_attr)

    S0 = _sc_gather_sum(Ps0, Pd0, src, dst)
    agg0 = _sc_msg_agg(Pm0, sig0, src, dst)
    e1, sig1 = _tc_edge_linear(e0, S0, el_W_e[0], el_b[0], want_sig=True)
    x1, Ps1, Pm1, Pd1, R1 = _tc_node_update(
        x0, R0, agg0[0], agg0[1],
        [el_W_src[1], nc_W_msg[1], el_W_dst[1], nc_W_root[1]],
        [zb, nc_b_msg[1], zb, nc_b_root[1]])

    S1 = _sc_gather_sum(Ps1, Pd1, src, dst)
    agg1 = _sc_msg_agg(Pm1, sig1, src, dst)
    e2, _ = _tc_edge_linear(e1, S1, el_W_e[1], el_b[1], want_sig=False)
    x2, Ps2, Pd2 = _tc_node_update(
        x1, R1, agg1[0], agg1[1],
        [el_W_src[2], el_W_dst[2]], [zb, zb])

    S2 = _sc_gather_sum(Ps2, Pd2, src, dst)
    ea0 = edge_attr[:, :1]
    W1a = mlp_W1[:U]
    w1b = mlp_W1[U:U + 1]
    logits = _tc_final(e2, S2, ea0, el_W_e[2], el_b[2], W1a, w1b, mlp_b1,
                       prelu_a, mlp_W2, mlp_b2)
    return logits


# speedup vs baseline: 2.8108x; 1.1611x over previous
import functools

import jax
import jax.numpy as jnp
from jax import lax
from jax.experimental import pallas as pl
from jax.experimental.pallas import tpu as pltpu
from jax.experimental.pallas import tpu_sc as plsc

N = 10000
E = 320000
U = 128

NC = 2
NS = 16
NW = NC * NS
PER_W = E // NW
T = 80
NT = PER_W // T
TM = 40
NTM = PER_W // TM
ZR = 80
NCH = N // ZR
CPS = -(-NCH // NS)

_mesh = functools.partial(
    plsc.VectorSubcoreMesh, core_axis_name="c", subcore_axis_name="s")


def _ring(nt, nb, fire, drain_out, finish):
    for u in range(nb - 1):
        fire(u, u % nb)
    n_iter = -(-nt // nb)

    def body(i, _):
        for j in range(nb):
            t = i * nb + j
            u = t + nb - 1
            b_u = (j + nb - 1) % nb

            @pl.when(t < nt)
            def _():
                finish(t, j)

            @pl.when((u >= nb) & (u < nt))
            def _():
                drain_out(u - nb, b_u)

            @pl.when(u < nt)
            def _():
                fire(u, b_u)
        return 0

    lax.fori_loop(0, n_iter, body, 0)
    for k in range(nb):
        drain_out(nt - nb + k, (nt - nb + k) % nb)


def _sc_msg_agg(Pm, sig, src3, dst3):
    @functools.partial(
        pl.kernel,
        mesh=_mesh(),
        out_type=jax.ShapeDtypeStruct((NC, N, U), jnp.float32),
        scratch_types=[
            pltpu.VMEM((PER_W,), jnp.int32),
            pltpu.VMEM((TM,), jnp.int32), pltpu.VMEM((TM,), jnp.int32),
            pltpu.VMEM((TM,), jnp.int32),
            pltpu.VMEM((TM, U), jnp.float32), pltpu.VMEM((TM, U), jnp.float32),
            pltpu.VMEM((TM, U), jnp.float32), pltpu.VMEM((TM, U), jnp.float32),
            pltpu.VMEM((TM, U), jnp.float32), pltpu.VMEM((TM, U), jnp.float32),
            pltpu.VMEM_SHARED((N, U), jnp.float32),
            pltpu.SemaphoreType.DMA, pltpu.SemaphoreType.DMA,
            pltpu.SemaphoreType.DMA, pltpu.SemaphoreType.DMA,
            pltpu.SemaphoreType.DMA, pltpu.SemaphoreType.DMA,
        ],
    )
    def k(pm_h, sig_h, src_h, dst_h, agg_out,
          six, dx0, dx1, dx2, g0, g1, g2, sg0, sg1, sg2, agg,
          si0, si1, si2, so0, so1, so2):
        cid = lax.axis_index("c")
        sid = lax.axis_index("s")
        wid = sid * NC + cid
        base0 = wid * PER_W
        G = (g0, g1, g2)
        DX = (dx0, dx1, dx2)
        SG = (sg0, sg1, sg2)
        SI = (si0, si1, si2)
        SO = (so0, so1, so2)

        pltpu.sync_copy(src_h.at[pl.ds(base0, PER_W)], six)

        def zrow(r, _):
            for cc in range(U // 16):
                g0[r, pl.ds(cc * 16, 16)] = jnp.zeros((16,), jnp.float32)
            return 0
        lax.fori_loop(0, TM, zrow, 0)
        for q in range(-(-(N // TM) // NS)):
            ch = sid + q * NS

            @pl.when(ch < N // TM)
            def _():
                pltpu.sync_copy(g0, agg.at[pl.ds(ch * TM, TM)])
        plsc.subcore_barrier()

        def fire(t, b):
            pltpu.async_copy(pm_h.at[six.at[pl.ds(t * TM, TM)]], G[b], SI[b])
            pltpu.async_copy(sig_h.at[pl.ds(base0 + t * TM, TM)], SG[b], SI[b])
            pltpu.async_copy(dst_h.at[pl.ds(base0 + t * TM, TM)], DX[b], SI[b])

        def drain_out(t, b):
            pltpu.make_async_copy(G[b], agg.at[DX[b]], SO[b]).wait()

        def finish(t, b):
            g, sg = G[b], SG[b]
            pltpu.make_async_copy(pm_h.at[six.at[pl.ds(t * TM, TM)]],
                                  g, SI[b]).wait()
            pltpu.make_async_copy(
                sig_h.at[pl.ds(base0 + t * TM, TM)], sg, SI[b]).wait()
            pltpu.make_async_copy(
                dst_h.at[pl.ds(base0 + t * TM, TM)], DX[b], SI[b]).wait()

            def row(r, _):
                for cc in range(U // 16):
                    c = cc * 16
                    g[r, pl.ds(c, 16)] = (g[r, pl.ds(c, 16)]
                                          * sg[r, pl.ds(c, 16)])
                return 0
            lax.fori_loop(0, TM, row, 0)
            pltpu.async_copy(g, agg.at[DX[b]], SO[b], add=True)

        _ring(NTM, 3, fire, drain_out, finish)
        plsc.subcore_barrier()

        for q in range(-(-(N // TM) // NS)):
            ch = sid + q * NS

            @pl.when(ch < N // TM)
            def _():
                pltpu.sync_copy(agg.at[pl.ds(ch * TM, TM)], g0)
                pltpu.sync_copy(g0, agg_out.at[cid, pl.ds(ch * TM, TM)])

    return k(Pm, sig, src3, dst3)


def _sc_gather_sum(Ps, Pd, src3, dst3):
    @functools.partial(
        pl.kernel,
        mesh=_mesh(),
        out_type=jax.ShapeDtypeStruct((E, U), jnp.float32),
        scratch_types=[
            pltpu.VMEM((PER_W,), jnp.int32), pltpu.VMEM((PER_W,), jnp.int32),
            pltpu.VMEM((T, U), jnp.float32), pltpu.VMEM((T, U), jnp.float32),
            pltpu.VMEM((T, U), jnp.float32), pltpu.VMEM((T, U), jnp.float32),
            pltpu.VMEM((T, U), jnp.float32), pltpu.VMEM((T, U), jnp.float32),
            pltpu.SemaphoreType.DMA, pltpu.SemaphoreType.DMA,
            pltpu.SemaphoreType.DMA, pltpu.SemaphoreType.DMA,
            pltpu.SemaphoreType.DMA, pltpu.SemaphoreType.DMA,
        ],
    )
    def k(ps_h, pd_h, src_h, dst_h, s_out,
          six, dix, g10, g11, g12, g20, g21, g22,
          si0, si1, si2, so0, so1, so2):
        wid = lax.axis_index("s") * NC + lax.axis_index("c")
        base0 = wid * PER_W
        G1 = (g10, g11, g12)
        G2 = (g20, g21, g22)
        SI = (si0, si1, si2)
        SO = (so0, so1, so2)

        pltpu.sync_copy(src_h.at[pl.ds(base0, PER_W)], six)
        pltpu.sync_copy(dst_h.at[pl.ds(base0, PER_W)], dix)

        def fire(t, b):
            pltpu.async_copy(ps_h.at[six.at[pl.ds(t * T, T)]], G1[b], SI[b])
            pltpu.async_copy(pd_h.at[dix.at[pl.ds(t * T, T)]], G2[b], SI[b])

        def drain_out(t, b):
            pltpu.make_async_copy(ps_h.at[pl.ds(0, T)], G2[b], SO[b]).wait()

        def finish(t, b):
            g1, g2 = G1[b], G2[b]
            pltpu.make_async_copy(ps_h.at[six.at[pl.ds(t * T, T)]],
                                  g1, SI[b]).wait()
            pltpu.make_async_copy(pd_h.at[dix.at[pl.ds(t * T, T)]],
                                  g2, SI[b]).wait()

            def row(r, _):
                for cc in range(U // 16):
                    c = cc * 16
                    g1[r, pl.ds(c, 16)] = (g1[r, pl.ds(c, 16)]
                                           + g2[r, pl.ds(c, 16)])
                return 0
            lax.fori_loop(0, T, row, 0)
            pltpu.async_copy(g1, s_out.at[pl.ds(base0 + t * T, T)], SO[b])

        _ring(NT, 3, fire, drain_out, finish)

    return k(Ps, Pd, src3, dst3)


EB = 2000


def _tc_node_init(pos, W_pos, b_pos, ws, bs):
    n_proj = len(ws)

    def body(pos_r, wp, bp, *rest):
        w_refs = rest[:n_proj]
        b_refs = rest[n_proj:2 * n_proj]
        outs = rest[2 * n_proj:]
        x = jnp.dot(pos_r[...], wp[...],
                    preferred_element_type=jnp.float32) + bp[...]
        outs[0][...] = x
        for wo, bo, oo in zip(w_refs, b_refs, outs[1:]):
            oo[...] = jnp.dot(x, wo[...],
                              preferred_element_type=jnp.float32) + bo[...]

    outs = (jax.ShapeDtypeStruct((N, U), jnp.float32),) + tuple(
        jax.ShapeDtypeStruct((N, w.shape[1]), jnp.float32) for w in ws)
    return pl.pallas_call(body, out_shape=outs)(
        pos, W_pos, b_pos.reshape(1, U), *ws,
        *[b.reshape(1, -1) for b in bs])


def _tc_node_update(x, R, agg0, agg1, ws, bs):
    n_proj = len(ws)

    def body(x_r, r_r, a0, a1, *rest):
        w_refs = rest[:n_proj]
        b_refs = rest[n_proj:2 * n_proj]
        outs = rest[2 * n_proj:]
        x1 = x_r[...] + jax.nn.relu(r_r[...] + a0[...] + a1[...])
        outs[0][...] = x1
        for wo, bo, oo in zip(w_refs, b_refs, outs[1:]):
            oo[...] = jnp.dot(x1, wo[...],
                              preferred_element_type=jnp.float32) + bo[...]

    outs = (jax.ShapeDtypeStruct((N, U), jnp.float32),) + tuple(
        jax.ShapeDtypeStruct((N, w.shape[1]), jnp.float32) for w in ws)
    return pl.pallas_call(body, out_shape=outs)(
        x, R, agg0, agg1, *ws, *[b.reshape(1, -1) for b in bs])


def _tc_edge_init(edge_attr, W_attr, b_attr):
    def body(ea, wa, ba, e_o, sig_o):
        e = jnp.dot(ea[...], wa[...],
                    preferred_element_type=jnp.float32) + ba[...]
        e_o[...] = e
        sig_o[...] = jax.nn.sigmoid(e)

    out = jax.ShapeDtypeStruct((E, U), jnp.float32)
    grid = (E // EB,)
    return pl.pallas_call(
        body,
        grid=grid,
        in_specs=[
            pl.BlockSpec((EB, 5), lambda i: (i, 0)),
            pl.BlockSpec((5, U), lambda i: (0, 0)),
            pl.BlockSpec((1, U), lambda i: (0, 0)),
        ],
        out_specs=(pl.BlockSpec((EB, U), lambda i: (i, 0)),) * 2,
        out_shape=(out, out),
    )(edge_attr, W_attr, b_attr.reshape(1, U))


def _tc_edge_linear(e, S, We, be, want_sig):
    def body(e_r, s_r, w, b, *outs):
        ev = e_r[...]
        enew = ev + jax.nn.relu(
            s_r[...] + jnp.dot(ev, w[...], preferred_element_type=jnp.float32)
            + b[...])
        outs[0][...] = enew
        if want_sig:
            outs[1][...] = jax.nn.sigmoid(enew)

    out = jax.ShapeDtypeStruct((E, U), jnp.float32)
    n_out = 2 if want_sig else 1
    grid = (E // EB,)
    res = pl.pallas_call(
        body,
        grid=grid,
        in_specs=[
            pl.BlockSpec((EB, U), lambda i: (i, 0)),
            pl.BlockSpec((EB, U), lambda i: (i, 0)),
            pl.BlockSpec((U, U), lambda i: (0, 0)),
            pl.BlockSpec((1, U), lambda i: (0, 0)),
        ],
        out_specs=(pl.BlockSpec((EB, U), lambda i: (i, 0)),) * n_out,
        out_shape=(out,) * n_out,
    )(e, S, We, be.reshape(1, U))
    return res if want_sig else (res[0], None)


def _tc_final(e, S, ea0, We, be, W1a, w1b, b1, a, W2, b2):
    H = W1a.shape[1]

    def body(e_r, s_r, ea_r, w, b, w1, w1b_r, b1_r, a_r, w2, b2_r, out_r):
        ev = e_r[...]
        ef = ev + jax.nn.relu(
            s_r[...] + jnp.dot(ev, w[...], preferred_element_type=jnp.float32)
            + b[...])
        h = (jnp.dot(ef, w1[...], preferred_element_type=jnp.float32)
             + ea_r[...] * w1b_r[...] + b1_r[...])
        h = jnp.where(h >= 0, h, a_r[...] * h)
        out_r[...] = jnp.dot(h, w2[...],
                             preferred_element_type=jnp.float32) + b2_r[...]

    grid = (E // EB,)
    return pl.pallas_call(
        body,
        grid=grid,
        in_specs=[
            pl.BlockSpec((EB, U), lambda i: (i, 0)),
            pl.BlockSpec((EB, U), lambda i: (i, 0)),
            pl.BlockSpec((EB, 1), lambda i: (i, 0)),
            pl.BlockSpec((U, U), lambda i: (0, 0)),
            pl.BlockSpec((1, U), lambda i: (0, 0)),
            pl.BlockSpec((U, H), lambda i: (0, 0)),
            pl.BlockSpec((1, H), lambda i: (0, 0)),
            pl.BlockSpec((1, H), lambda i: (0, 0)),
            pl.BlockSpec((1, 1), lambda i: (0, 0)),
            pl.BlockSpec((H, 1), lambda i: (0, 0)),
            pl.BlockSpec((1, 1), lambda i: (0, 0)),
        ],
        out_specs=pl.BlockSpec((EB, 1), lambda i: (i, 0)),
        out_shape=jax.ShapeDtypeStruct((E, 1), jnp.float32),
    )(e, S, ea0, We, be.reshape(1, U), W1a, w1b, b1.reshape(1, H),
      a.reshape(1, 1), W2, b2.reshape(1, 1))


def kernel(pos, edge_attr, edge_index, W_pos, b_pos, W_attr, b_attr,
           nc_W_root, nc_b_root, nc_W_msg, nc_b_msg, el_W_src, el_W_dst,
           el_W_e, el_b, mlp_W1, mlp_b1, prelu_a, mlp_W2, mlp_b2):
    src = edge_index[0]
    dst = edge_index[1]
    zb = jnp.zeros((U,), jnp.float32)

    x0, Ps0, Pm0, Pd0, R0 = _tc_node_init(
        pos, W_pos, b_pos,
        [el_W_src[0], nc_W_msg[0], el_W_dst[0], nc_W_root[0]],
        [zb, nc_b_msg[0], zb, nc_b_root[0]])
    e0, sig0 = _tc_edge_init(edge_attr, W_attr, b_attr)

    S0 = _sc_gather_sum(Ps0, Pd0, src, dst)
    agg0 = _sc_msg_agg(Pm0, sig0, src, dst)
    e1, sig1 = _tc_edge_linear(e0, S0, el_W_e[0], el_b[0], want_sig=True)
    x1, Ps1, Pm1, Pd1, R1 = _tc_node_update(
        x0, R0, agg0[0], agg0[1],
        [el_W_src[1], nc_W_msg[1], el_W_dst[1], nc_W_root[1]],
        [zb, nc_b_msg[1], zb, nc_b_root[1]])

    S1 = _sc_gather_sum(Ps1, Pd1, src, dst)
    agg1 = _sc_msg_agg(Pm1, sig1, src, dst)
    e2, _ = _tc_edge_linear(e1, S1, el_W_e[1], el_b[1], want_sig=False)
    x2, Ps2, Pd2 = _tc_node_update(
        x1, R1, agg1[0], agg1[1],
        [el_W_src[2], el_W_dst[2]], [zb, zb])

    S2 = _sc_gather_sum(Ps2, Pd2, src, dst)
    ea0 = edge_attr[:, :1]
    W1a = mlp_W1[:U]
    w1b = mlp_W1[U:U + 1]
    logits = _tc_final(e2, S2, ea0, el_W_e[2], el_b[2], W1a, w1b, mlp_b1,
                       prelu_a, mlp_W2, mlp_b2)
    return logits


# speedup vs baseline: 2.9239x; 1.0402x over previous
import functools

import jax
import jax.numpy as jnp
from jax import lax
from jax.experimental import pallas as pl
from jax.experimental.pallas import tpu as pltpu
from jax.experimental.pallas import tpu_sc as plsc

N = 10000
E = 320000
U = 128

NC = 2
NS = 16
NW = NC * NS
PER_W = E // NW
T = 80
NT = PER_W // T
TM = 40
NTM = PER_W // TM
ZR = 80
NCH = N // ZR
CPS = -(-NCH // NS)

_mesh = functools.partial(
    plsc.VectorSubcoreMesh, core_axis_name="c", subcore_axis_name="s")


def _ring(nt, nb, fire, drain_out, finish):
    for u in range(nb - 1):
        fire(u, u % nb)
    n_iter = -(-nt // nb)

    def body(i, _):
        for j in range(nb):
            t = i * nb + j
            u = t + nb - 1
            b_u = (j + nb - 1) % nb

            @pl.when(t < nt)
            def _():
                finish(t, j)

            @pl.when((u >= nb) & (u < nt))
            def _():
                drain_out(u - nb, b_u)

            @pl.when(u < nt)
            def _():
                fire(u, b_u)
        return 0

    lax.fori_loop(0, n_iter, body, 0)
    for k in range(nb):
        drain_out(nt - nb + k, (nt - nb + k) % nb)


def _sc_msg_agg(Pm, sig, src3, dst3):
    @functools.partial(
        pl.kernel,
        mesh=_mesh(),
        out_type=jax.ShapeDtypeStruct((NC, N, U), jnp.float32),
        scratch_types=[
            pltpu.VMEM((PER_W,), jnp.int32),
            pltpu.VMEM((TM,), jnp.int32), pltpu.VMEM((TM,), jnp.int32),
            pltpu.VMEM((TM,), jnp.int32),
            pltpu.VMEM((TM, U), jnp.float32), pltpu.VMEM((TM, U), jnp.float32),
            pltpu.VMEM((TM, U), jnp.float32), pltpu.VMEM((TM, U), jnp.float32),
            pltpu.VMEM((TM, U), jnp.float32), pltpu.VMEM((TM, U), jnp.float32),
            pltpu.VMEM_SHARED((N, U), jnp.float32),
            pltpu.SemaphoreType.DMA, pltpu.SemaphoreType.DMA,
            pltpu.SemaphoreType.DMA, pltpu.SemaphoreType.DMA,
            pltpu.SemaphoreType.DMA, pltpu.SemaphoreType.DMA,
        ],
    )
    def k(pm_h, sig_h, src_h, dst_h, agg_out,
          six, dx0, dx1, dx2, g0, g1, g2, sg0, sg1, sg2, agg,
          si0, si1, si2, so0, so1, so2):
        cid = lax.axis_index("c")
        sid = lax.axis_index("s")
        wid = sid * NC + cid
        base0 = wid * PER_W
        G = (g0, g1, g2)
        DX = (dx0, dx1, dx2)
        SG = (sg0, sg1, sg2)
        SI = (si0, si1, si2)
        SO = (so0, so1, so2)

        pltpu.sync_copy(src_h.at[pl.ds(base0, PER_W)], six)

        def zrow(r, _):
            for cc in range(U // 16):
                g0[r, pl.ds(cc * 16, 16)] = jnp.zeros((16,), jnp.float32)
            return 0
        lax.fori_loop(0, TM, zrow, 0)
        for q in range(-(-(N // TM) // NS)):
            ch = sid + q * NS

            @pl.when(ch < N // TM)
            def _():
                pltpu.sync_copy(g0, agg.at[pl.ds(ch * TM, TM)])
        plsc.subcore_barrier()

        def fire(t, b):
            pltpu.async_copy(pm_h.at[six.at[pl.ds(t * TM, TM)]], G[b], SI[b])
            pltpu.async_copy(sig_h.at[pl.ds(base0 + t * TM, TM)], SG[b], SI[b])
            pltpu.async_copy(dst_h.at[pl.ds(base0 + t * TM, TM)], DX[b], SI[b])

        def drain_out(t, b):
            pltpu.make_async_copy(G[b], agg.at[DX[b]], SO[b]).wait()

        def finish(t, b):
            g, sg = G[b], SG[b]
            pltpu.make_async_copy(pm_h.at[six.at[pl.ds(t * TM, TM)]],
                                  g, SI[b]).wait()
            pltpu.make_async_copy(
                sig_h.at[pl.ds(base0 + t * TM, TM)], sg, SI[b]).wait()
            pltpu.make_async_copy(
                dst_h.at[pl.ds(base0 + t * TM, TM)], DX[b], SI[b]).wait()

            def row(r, _):
                for cc in range(U // 16):
                    c = cc * 16
                    g[r, pl.ds(c, 16)] = (g[r, pl.ds(c, 16)]
                                          * sg[r, pl.ds(c, 16)])
                return 0
            lax.fori_loop(0, TM, row, 0)
            pltpu.async_copy(g, agg.at[DX[b]], SO[b], add=True)

        _ring(NTM, 3, fire, drain_out, finish)
        plsc.subcore_barrier()

        for q in range(-(-(N // TM) // NS)):
            ch = sid + q * NS

            @pl.when(ch < N // TM)
            def _():
                pltpu.sync_copy(agg.at[pl.ds(ch * TM, TM)], g0)
                pltpu.sync_copy(g0, agg_out.at[cid, pl.ds(ch * TM, TM)])

    return k(Pm, sig, src3, dst3)


def _sc_gather_sum(Ps, Pd, src3, dst3):
    @functools.partial(
        pl.kernel,
        mesh=_mesh(),
        out_type=jax.ShapeDtypeStruct((E, U), jnp.float32),
        scratch_types=[
            pltpu.VMEM((PER_W,), jnp.int32), pltpu.VMEM((PER_W,), jnp.int32),
            pltpu.VMEM((T, U), jnp.float32), pltpu.VMEM((T, U), jnp.float32),
            pltpu.VMEM((T, U), jnp.float32), pltpu.VMEM((T, U), jnp.float32),
            pltpu.VMEM((T, U), jnp.float32), pltpu.VMEM((T, U), jnp.float32),
            pltpu.SemaphoreType.DMA, pltpu.SemaphoreType.DMA,
            pltpu.SemaphoreType.DMA, pltpu.SemaphoreType.DMA,
            pltpu.SemaphoreType.DMA, pltpu.SemaphoreType.DMA,
        ],
    )
    def k(ps_h, pd_h, src_h, dst_h, s_out,
          six, dix, g10, g11, g12, g20, g21, g22,
          si0, si1, si2, so0, so1, so2):
        wid = lax.axis_index("s") * NC + lax.axis_index("c")
        base0 = wid * PER_W
        G1 = (g10, g11, g12)
        G2 = (g20, g21, g22)
        SI = (si0, si1, si2)
        SO = (so0, so1, so2)

        pltpu.sync_copy(src_h.at[pl.ds(base0, PER_W)], six)
        pltpu.sync_copy(dst_h.at[pl.ds(base0, PER_W)], dix)

        def fire(t, b):
            pltpu.async_copy(ps_h.at[six.at[pl.ds(t * T, T)]], G1[b], SI[b])
            pltpu.async_copy(pd_h.at[dix.at[pl.ds(t * T, T)]], G2[b], SI[b])

        def drain_out(t, b):
            pltpu.make_async_copy(ps_h.at[pl.ds(0, T)], G2[b], SO[b]).wait()

        def finish(t, b):
            g1, g2 = G1[b], G2[b]
            pltpu.make_async_copy(ps_h.at[six.at[pl.ds(t * T, T)]],
                                  g1, SI[b]).wait()
            pltpu.make_async_copy(pd_h.at[dix.at[pl.ds(t * T, T)]],
                                  g2, SI[b]).wait()

            def row(r, _):
                for cc in range(U // 16):
                    c = cc * 16
                    g1[r, pl.ds(c, 16)] = (g1[r, pl.ds(c, 16)]
                                           + g2[r, pl.ds(c, 16)])
                return 0
            lax.fori_loop(0, T, row, 0)
            pltpu.async_copy(g1, s_out.at[pl.ds(base0 + t * T, T)], SO[b])

        _ring(NT, 3, fire, drain_out, finish)

    return k(Ps, Pd, src3, dst3)


EB = 2000


def _tc_node_init(pos, W_pos, b_pos, ws, bs):
    n_proj = len(ws)

    def body(pos_r, wp, bp, *rest):
        w_refs = rest[:n_proj]
        b_refs = rest[n_proj:2 * n_proj]
        outs = rest[2 * n_proj:]
        x = jnp.dot(pos_r[...], wp[...],
                    preferred_element_type=jnp.float32) + bp[...]
        outs[0][...] = x
        for wo, bo, oo in zip(w_refs, b_refs, outs[1:]):
            oo[...] = jnp.dot(x, wo[...],
                              preferred_element_type=jnp.float32) + bo[...]

    outs = (jax.ShapeDtypeStruct((N, U), jnp.float32),) + tuple(
        jax.ShapeDtypeStruct((N, w.shape[1]), jnp.float32) for w in ws)
    return pl.pallas_call(body, out_shape=outs)(
        pos, W_pos, b_pos.reshape(1, U), *ws,
        *[b.reshape(1, -1) for b in bs])


def _tc_node_update(x, R, agg0, agg1, ws, bs):
    n_proj = len(ws)

    def body(x_r, r_r, a0, a1, *rest):
        w_refs = rest[:n_proj]
        b_refs = rest[n_proj:2 * n_proj]
        outs = rest[2 * n_proj:]
        x1 = x_r[...] + jax.nn.relu(r_r[...] + a0[...] + a1[...])
        outs[0][...] = x1
        for wo, bo, oo in zip(w_refs, b_refs, outs[1:]):
            oo[...] = jnp.dot(x1, wo[...],
                              preferred_element_type=jnp.float32) + bo[...]

    outs = (jax.ShapeDtypeStruct((N, U), jnp.float32),) + tuple(
        jax.ShapeDtypeStruct((N, w.shape[1]), jnp.float32) for w in ws)
    return pl.pallas_call(body, out_shape=outs)(
        x, R, agg0, agg1, *ws, *[b.reshape(1, -1) for b in bs])


def _tc_edge_init(edge_attr, W_attr, b_attr):
    def body(ea, wa, ba, e_o, sig_o):
        e = jnp.dot(ea[...], wa[...],
                    preferred_element_type=jnp.float32) + ba[...]
        e_o[...] = e.astype(jnp.bfloat16)
        sig_o[...] = jax.nn.sigmoid(e)

    grid = (E // EB,)
    return pl.pallas_call(
        body,
        grid=grid,
        in_specs=[
            pl.BlockSpec((EB, 5), lambda i: (i, 0)),
            pl.BlockSpec((5, U), lambda i: (0, 0)),
            pl.BlockSpec((1, U), lambda i: (0, 0)),
        ],
        out_specs=(pl.BlockSpec((EB, U), lambda i: (i, 0)),) * 2,
        out_shape=(jax.ShapeDtypeStruct((E, U), jnp.bfloat16),
                   jax.ShapeDtypeStruct((E, U), jnp.float32)),
    )(edge_attr, W_attr, b_attr.reshape(1, U))


def _tc_edge_linear(e, S, We, be, want_sig):
    def body(e_r, s_r, w, b, *outs):
        ev = e_r[...].astype(jnp.float32)
        enew = ev + jax.nn.relu(
            s_r[...] + jnp.dot(e_r[...], w[...],
                               preferred_element_type=jnp.float32)
            + b[...])
        outs[0][...] = enew.astype(jnp.bfloat16)
        if want_sig:
            outs[1][...] = jax.nn.sigmoid(enew)

    n_out = 2 if want_sig else 1
    grid = (E // EB,)
    res = pl.pallas_call(
        body,
        grid=grid,
        in_specs=[
            pl.BlockSpec((EB, U), lambda i: (i, 0)),
            pl.BlockSpec((EB, U), lambda i: (i, 0)),
            pl.BlockSpec((U, U), lambda i: (0, 0)),
            pl.BlockSpec((1, U), lambda i: (0, 0)),
        ],
        out_specs=(pl.BlockSpec((EB, U), lambda i: (i, 0)),) * n_out,
        out_shape=(jax.ShapeDtypeStruct((E, U), jnp.bfloat16),
                   jax.ShapeDtypeStruct((E, U), jnp.float32))[:n_out],
    )(e, S, We, be.reshape(1, U))
    return res if want_sig else (res[0], None)


def _tc_final(e, S, ea0, We, be, W1a, w1b, b1, a, W2, b2):
    H = W1a.shape[1]

    def body(e_r, s_r, ea_r, w, b, w1, w1b_r, b1_r, a_r, w2, b2_r, out_r):
        ev = e_r[...].astype(jnp.float32)
        ef = ev + jax.nn.relu(
            s_r[...] + jnp.dot(e_r[...], w[...],
                               preferred_element_type=jnp.float32)
            + b[...])
        h = (jnp.dot(ef, w1[...], preferred_element_type=jnp.float32)
             + ea_r[...] * w1b_r[...] + b1_r[...])
        h = jnp.where(h >= 0, h, a_r[...] * h)
        out_r[...] = jnp.dot(h, w2[...],
                             preferred_element_type=jnp.float32) + b2_r[...]

    grid = (E // EB,)
    return pl.pallas_call(
        body,
        grid=grid,
        in_specs=[
            pl.BlockSpec((EB, U), lambda i: (i, 0)),
            pl.BlockSpec((EB, U), lambda i: (i, 0)),
            pl.BlockSpec((EB, 1), lambda i: (i, 0)),
            pl.BlockSpec((U, U), lambda i: (0, 0)),
            pl.BlockSpec((1, U), lambda i: (0, 0)),
            pl.BlockSpec((U, H), lambda i: (0, 0)),
            pl.BlockSpec((1, H), lambda i: (0, 0)),
            pl.BlockSpec((1, H), lambda i: (0, 0)),
            pl.BlockSpec((1, 1), lambda i: (0, 0)),
            pl.BlockSpec((H, 1), lambda i: (0, 0)),
            pl.BlockSpec((1, 1), lambda i: (0, 0)),
        ],
        out_specs=pl.BlockSpec((EB, 1), lambda i: (i, 0)),
        out_shape=jax.ShapeDtypeStruct((E, 1), jnp.float32),
    )(e, S, ea0, We, be.reshape(1, U), W1a, w1b, b1.reshape(1, H),
      a.reshape(1, 1), W2, b2.reshape(1, 1))


def kernel(pos, edge_attr, edge_index, W_pos, b_pos, W_attr, b_attr,
           nc_W_root, nc_b_root, nc_W_msg, nc_b_msg, el_W_src, el_W_dst,
           el_W_e, el_b, mlp_W1, mlp_b1, prelu_a, mlp_W2, mlp_b2):
    src = edge_index[0]
    dst = edge_index[1]
    zb = jnp.zeros((U,), jnp.float32)

    x0, Ps0, Pm0, Pd0, R0 = _tc_node_init(
        pos, W_pos, b_pos,
        [el_W_src[0], nc_W_msg[0], el_W_dst[0], nc_W_root[0]],
        [zb, nc_b_msg[0], zb, nc_b_root[0]])
    e0, sig0 = _tc_edge_init(edge_attr, W_attr, b_attr)

    S0 = _sc_gather_sum(Ps0, Pd0, src, dst)
    agg0 = _sc_msg_agg(Pm0, sig0, src, dst)
    e1, sig1 = _tc_edge_linear(e0, S0, el_W_e[0], el_b[0], want_sig=True)
    x1, Ps1, Pm1, Pd1, R1 = _tc_node_update(
        x0, R0, agg0[0], agg0[1],
        [el_W_src[1], nc_W_msg[1], el_W_dst[1], nc_W_root[1]],
        [zb, nc_b_msg[1], zb, nc_b_root[1]])

    S1 = _sc_gather_sum(Ps1, Pd1, src, dst)
    agg1 = _sc_msg_agg(Pm1, sig1, src, dst)
    e2, _ = _tc_edge_linear(e1, S1, el_W_e[1], el_b[1], want_sig=False)
    x2, Ps2, Pd2 = _tc_node_update(
        x1, R1, agg1[0], agg1[1],
        [el_W_src[2], el_W_dst[2]], [zb, zb])

    S2 = _sc_gather_sum(Ps2, Pd2, src, dst)
    ea0 = edge_attr[:, :1]
    W1a = mlp_W1[:U]
    w1b = mlp_W1[U:U + 1]
    logits = _tc_final(e2, S2, ea0, el_W_e[2], el_b[2], W1a, w1b, mlp_b1,
                       prelu_a, mlp_W2, mlp_b2)
    return logits


# speedup vs baseline: 2.9480x; 1.0082x over previous
import functools

import jax
import jax.numpy as jnp
from jax import lax
from jax.experimental import pallas as pl
from jax.experimental.pallas import tpu as pltpu
from jax.experimental.pallas import tpu_sc as plsc

N = 10000
E = 320000
U = 128

NC = 2
NS = 16
NW = NC * NS
PER_W = E // NW
T = 80
NT = PER_W // T
TM = 40
NTM = PER_W // TM

_mesh = functools.partial(
    plsc.VectorSubcoreMesh, core_axis_name="c", subcore_axis_name="s")


def _ring(nt, nb, fire, drain_out, finish):
    for u in range(nb - 1):
        fire(u, u % nb)
    n_iter = -(-nt // nb)

    def body(i, _):
        for j in range(nb):
            t = i * nb + j
            u = t + nb - 1
            b_u = (j + nb - 1) % nb

            @pl.when(t < nt)
            def _():
                finish(t, j)

            @pl.when((u >= nb) & (u < nt))
            def _():
                drain_out(u - nb, b_u)

            @pl.when(u < nt)
            def _():
                fire(u, b_u)
        return 0

    lax.fori_loop(0, n_iter, body, 0)
    for k in range(nb):
        drain_out(nt - nb + k, (nt - nb + k) % nb)


def _sc_msg_agg(Pm, sig, src, dst):
    @functools.partial(
        pl.kernel,
        mesh=_mesh(),
        out_type=jax.ShapeDtypeStruct((NC, N, U), jnp.float32),
        scratch_types=[
            pltpu.VMEM((PER_W,), jnp.int32),
            pltpu.VMEM((TM,), jnp.int32), pltpu.VMEM((TM,), jnp.int32),
            pltpu.VMEM((TM,), jnp.int32),
            pltpu.VMEM((TM, U), jnp.float32), pltpu.VMEM((TM, U), jnp.float32),
            pltpu.VMEM((TM, U), jnp.float32),
            pltpu.VMEM((TM, U), jnp.float32), pltpu.VMEM((TM, U), jnp.float32),
            pltpu.VMEM((TM, U), jnp.float32),
            pltpu.VMEM_SHARED((N, U), jnp.float32),
            pltpu.SemaphoreType.DMA, pltpu.SemaphoreType.DMA,
            pltpu.SemaphoreType.DMA, pltpu.SemaphoreType.DMA,
            pltpu.SemaphoreType.DMA, pltpu.SemaphoreType.DMA,
        ],
    )
    def k(pm_h, sig_h, src_h, dst_h, agg_out,
          six, dx0, dx1, dx2, g0, g1, g2, sg0, sg1, sg2, agg,
          si0, si1, si2, so0, so1, so2):
        cid = lax.axis_index("c")
        sid = lax.axis_index("s")
        wid = sid * NC + cid
        base0 = wid * PER_W
        G = (g0, g1, g2)
        DX = (dx0, dx1, dx2)
        SG = (sg0, sg1, sg2)
        SI = (si0, si1, si2)
        SO = (so0, so1, so2)

        pltpu.sync_copy(src_h.at[pl.ds(base0, PER_W)], six)

        def zrow(r, _):
            for cc in range(U // 16):
                g0[r, pl.ds(cc * 16, 16)] = jnp.zeros((16,), jnp.float32)
            return 0
        lax.fori_loop(0, TM, zrow, 0)
        for q in range(-(-(N // TM) // NS)):
            ch = sid + q * NS

            @pl.when(ch < N // TM)
            def _():
                pltpu.sync_copy(g0, agg.at[pl.ds(ch * TM, TM)])
        plsc.subcore_barrier()

        def fire(t, b):
            pltpu.async_copy(pm_h.at[six.at[pl.ds(t * TM, TM)]], G[b], SI[b])
            pltpu.async_copy(sig_h.at[pl.ds(base0 + t * TM, TM)], SG[b], SI[b])
            pltpu.async_copy(dst_h.at[pl.ds(base0 + t * TM, TM)], DX[b], SI[b])

        def drain_out(t, b):
            pltpu.make_async_copy(G[b], agg.at[DX[b]], SO[b]).wait()

        def finish(t, b):
            g, sg = G[b], SG[b]
            pltpu.make_async_copy(pm_h.at[six.at[pl.ds(t * TM, TM)]],
                                  g, SI[b]).wait()
            pltpu.make_async_copy(
                sig_h.at[pl.ds(base0 + t * TM, TM)], sg, SI[b]).wait()
            pltpu.make_async_copy(
                dst_h.at[pl.ds(base0 + t * TM, TM)], DX[b], SI[b]).wait()

            def row(r, _):
                for cc in range(U // 16):
                    c = cc * 16
                    g[r, pl.ds(c, 16)] = (g[r, pl.ds(c, 16)]
                                          * sg[r, pl.ds(c, 16)])
                return 0
            lax.fori_loop(0, TM, row, 0)
            pltpu.async_copy(g, agg.at[DX[b]], SO[b], add=True)

        _ring(NTM, 3, fire, drain_out, finish)
        plsc.subcore_barrier()

        for q in range(-(-(N // TM) // NS)):
            ch = sid + q * NS

            @pl.when(ch < N // TM)
            def _():
                pltpu.sync_copy(agg.at[pl.ds(ch * TM, TM)], g0)
                pltpu.sync_copy(g0, agg_out.at[cid, pl.ds(ch * TM, TM)])

    return k(Pm, sig, src, dst)


def _sc_gather_sum(Ps, Pd, src, dst):
    @functools.partial(
        pl.kernel,
        mesh=_mesh(),
        out_type=jax.ShapeDtypeStruct((E, U), jnp.float32),
        scratch_types=[
            pltpu.VMEM((PER_W,), jnp.int32), pltpu.VMEM((PER_W,), jnp.int32),
            pltpu.VMEM((T, U), jnp.float32), pltpu.VMEM((T, U), jnp.float32),
            pltpu.VMEM((T, U), jnp.float32), pltpu.VMEM((T, U), jnp.float32),
            pltpu.VMEM((T, U), jnp.float32), pltpu.VMEM((T, U), jnp.float32),
            pltpu.VMEM((T, U), jnp.float32), pltpu.VMEM((T, U), jnp.float32),
            pltpu.SemaphoreType.DMA, pltpu.SemaphoreType.DMA,
            pltpu.SemaphoreType.DMA, pltpu.SemaphoreType.DMA,
            pltpu.SemaphoreType.DMA, pltpu.SemaphoreType.DMA,
            pltpu.SemaphoreType.DMA, pltpu.SemaphoreType.DMA,
        ],
    )
    def k(ps_h, pd_h, src_h, dst_h, s_out,
          six, dix, g10, g11, g12, g13, g20, g21, g22, g23,
          si0, si1, si2, si3, so0, so1, so2, so3):
        wid = lax.axis_index("s") * NC + lax.axis_index("c")
        base0 = wid * PER_W
        G1 = (g10, g11, g12, g13)
        G2 = (g20, g21, g22, g23)
        SI = (si0, si1, si2, si3)
        SO = (so0, so1, so2, so3)

        pltpu.sync_copy(src_h.at[pl.ds(base0, PER_W)], six)
        pltpu.sync_copy(dst_h.at[pl.ds(base0, PER_W)], dix)

        def fire(t, b):
            pltpu.async_copy(ps_h.at[six.at[pl.ds(t * T, T)]], G1[b], SI[b])
            pltpu.async_copy(pd_h.at[dix.at[pl.ds(t * T, T)]], G2[b], SI[b])

        def drain_out(t, b):
            pltpu.make_async_copy(ps_h.at[pl.ds(0, T)], G2[b], SO[b]).wait()

        def finish(t, b):
            g1, g2 = G1[b], G2[b]
            pltpu.make_async_copy(ps_h.at[six.at[pl.ds(t * T, T)]],
                                  g1, SI[b]).wait()
            pltpu.make_async_copy(pd_h.at[dix.at[pl.ds(t * T, T)]],
                                  g2, SI[b]).wait()

            def row(r, _):
                for cc in range(U // 16):
                    c = cc * 16
                    g1[r, pl.ds(c, 16)] = (g1[r, pl.ds(c, 16)]
                                           + g2[r, pl.ds(c, 16)])
                return 0
            lax.fori_loop(0, T, row, 0)
            pltpu.async_copy(g1, s_out.at[pl.ds(base0 + t * T, T)], SO[b])

        _ring(NT, 4, fire, drain_out, finish)

    return k(Ps, Pd, src, dst)


EB = 2000


def _tc_node_init(pos, W_pos, b_pos, ws, bs):
    n_proj = len(ws)

    def body(pos_r, wp, bp, *rest):
        w_refs = rest[:n_proj]
        b_refs = rest[n_proj:2 * n_proj]
        outs = rest[2 * n_proj:]
        x = jnp.dot(pos_r[...], wp[...],
                    preferred_element_type=jnp.float32) + bp[...]
        outs[0][...] = x
        for wo, bo, oo in zip(w_refs, b_refs, outs[1:]):
            oo[...] = jnp.dot(x, wo[...],
                              preferred_element_type=jnp.float32) + bo[...]

    outs = (jax.ShapeDtypeStruct((N, U), jnp.float32),) + tuple(
        jax.ShapeDtypeStruct((N, w.shape[1]), jnp.float32) for w in ws)
    return pl.pallas_call(body, out_shape=outs)(
        pos, W_pos, b_pos.reshape(1, U), *ws,
        *[b.reshape(1, -1) for b in bs])


def _tc_node_update(x, R, agg0, agg1, ws, bs):
    n_proj = len(ws)

    def body(x_r, r_r, a0, a1, *rest):
        w_refs = rest[:n_proj]
        b_refs = rest[n_proj:2 * n_proj]
        outs = rest[2 * n_proj:]
        x1 = x_r[...] + jax.nn.relu(r_r[...] + a0[...] + a1[...])
        outs[0][...] = x1
        for wo, bo, oo in zip(w_refs, b_refs, outs[1:]):
            oo[...] = jnp.dot(x1, wo[...],
                              preferred_element_type=jnp.float32) + bo[...]

    outs = (jax.ShapeDtypeStruct((N, U), jnp.float32),) + tuple(
        jax.ShapeDtypeStruct((N, w.shape[1]), jnp.float32) for w in ws)
    return pl.pallas_call(body, out_shape=outs)(
        x, R, agg0, agg1, *ws, *[b.reshape(1, -1) for b in bs])


def _tc_edge_init(edge_attr, W_attr, b_attr):
    def body(ea, wa, ba, e_o, sig_o):
        e = jnp.dot(ea[...], wa[...],
                    preferred_element_type=jnp.float32) + ba[...]
        e_o[...] = e.astype(jnp.bfloat16)
        sig_o[...] = jax.nn.sigmoid(e)

    grid = (E // EB,)
    return pl.pallas_call(
        body,
        grid=grid,
        in_specs=[
            pl.BlockSpec((EB, 5), lambda i: (i, 0)),
            pl.BlockSpec((5, U), lambda i: (0, 0)),
            pl.BlockSpec((1, U), lambda i: (0, 0)),
        ],
        out_specs=(pl.BlockSpec((EB, U), lambda i: (i, 0)),) * 2,
        out_shape=(jax.ShapeDtypeStruct((E, U), jnp.bfloat16),
                   jax.ShapeDtypeStruct((E, U), jnp.float32)),
    )(edge_attr, W_attr, b_attr.reshape(1, U))


def _tc_edge_linear(e, S, We, be, want_sig):
    def body(e_r, s_r, w, b, *outs):
        ev = e_r[...].astype(jnp.float32)
        enew = ev + jax.nn.relu(
            s_r[...] + jnp.dot(e_r[...], w[...],
                               preferred_element_type=jnp.float32)
            + b[...])
        outs[0][...] = enew.astype(jnp.bfloat16)
        if want_sig:
            outs[1][...] = jax.nn.sigmoid(enew)

    n_out = 2 if want_sig else 1
    grid = (E // EB,)
    res = pl.pallas_call(
        body,
        grid=grid,
        in_specs=[
            pl.BlockSpec((EB, U), lambda i: (i, 0)),
            pl.BlockSpec((EB, U), lambda i: (i, 0)),
            pl.BlockSpec((U, U), lambda i: (0, 0)),
            pl.BlockSpec((1, U), lambda i: (0, 0)),
        ],
        out_specs=(pl.BlockSpec((EB, U), lambda i: (i, 0)),) * n_out,
        out_shape=(jax.ShapeDtypeStruct((E, U), jnp.bfloat16),
                   jax.ShapeDtypeStruct((E, U), jnp.float32))[:n_out],
    )(e, S, We, be.reshape(1, U))
    return res if want_sig else (res[0], None)


def _tc_final(e, S, ea0, We, be, W1a, w1b, b1, a, W2, b2):
    H = W1a.shape[1]

    def body(e_r, s_r, ea_r, w, b, w1, w1b_r, b1_r, a_r, w2, b2_r, out_r):
        ev = e_r[...].astype(jnp.float32)
        ef = ev + jax.nn.relu(
            s_r[...] + jnp.dot(e_r[...], w[...],
                               preferred_element_type=jnp.float32)
            + b[...])
        h = (jnp.dot(ef, w1[...], preferred_element_type=jnp.float32)
             + ea_r[...] * w1b_r[...] + b1_r[...])
        h = jnp.where(h >= 0, h, a_r[...] * h)
        out_r[...] = jnp.dot(h, w2[...],
                             preferred_element_type=jnp.float32) + b2_r[...]

    grid = (E // EB,)
    return pl.pallas_call(
        body,
        grid=grid,
        in_specs=[
            pl.BlockSpec((EB, U), lambda i: (i, 0)),
            pl.BlockSpec((EB, U), lambda i: (i, 0)),
            pl.BlockSpec((EB, 1), lambda i: (i, 0)),
            pl.BlockSpec((U, U), lambda i: (0, 0)),
            pl.BlockSpec((1, U), lambda i: (0, 0)),
            pl.BlockSpec((U, H), lambda i: (0, 0)),
            pl.BlockSpec((1, H), lambda i: (0, 0)),
            pl.BlockSpec((1, H), lambda i: (0, 0)),
            pl.BlockSpec((1, 1), lambda i: (0, 0)),
            pl.BlockSpec((H, 1), lambda i: (0, 0)),
            pl.BlockSpec((1, 1), lambda i: (0, 0)),
        ],
        out_specs=pl.BlockSpec((EB, 1), lambda i: (i, 0)),
        out_shape=jax.ShapeDtypeStruct((E, 1), jnp.float32),
    )(e, S, ea0, We, be.reshape(1, U), W1a, w1b, b1.reshape(1, H),
      a.reshape(1, 1), W2, b2.reshape(1, 1))


def kernel(pos, edge_attr, edge_index, W_pos, b_pos, W_attr, b_attr,
           nc_W_root, nc_b_root, nc_W_msg, nc_b_msg, el_W_src, el_W_dst,
           el_W_e, el_b, mlp_W1, mlp_b1, prelu_a, mlp_W2, mlp_b2):
    src = edge_index[0]
    dst = edge_index[1]
    zb = jnp.zeros((U,), jnp.float32)

    x0, Ps0, Pm0, Pd0, R0 = _tc_node_init(
        pos, W_pos, b_pos,
        [el_W_src[0], nc_W_msg[0], el_W_dst[0], nc_W_root[0]],
        [zb, nc_b_msg[0], zb, nc_b_root[0]])
    e0, sig0 = _tc_edge_init(edge_attr, W_attr, b_attr)

    S0 = _sc_gather_sum(Ps0, Pd0, src, dst)
    agg0 = _sc_msg_agg(Pm0, sig0, src, dst)
    e1, sig1 = _tc_edge_linear(e0, S0, el_W_e[0], el_b[0], want_sig=True)
    x1, Ps1, Pm1, Pd1, R1 = _tc_node_update(
        x0, R0, agg0[0], agg0[1],
        [el_W_src[1], nc_W_msg[1], el_W_dst[1], nc_W_root[1]],
        [zb, nc_b_msg[1], zb, nc_b_root[1]])

    S1 = _sc_gather_sum(Ps1, Pd1, src, dst)
    agg1 = _sc_msg_agg(Pm1, sig1, src, dst)
    e2, _ = _tc_edge_linear(e1, S1, el_W_e[1], el_b[1], want_sig=False)
    x2, Ps2, Pd2 = _tc_node_update(
        x1, R1, agg1[0], agg1[1],
        [el_W_src[2], el_W_dst[2]], [zb, zb])

    S2 = _sc_gather_sum(Ps2, Pd2, src, dst)
    ea0 = edge_attr[:, :1]
    W1a = mlp_W1[:U]
    w1b = mlp_W1[U:U + 1]
    logits = _tc_final(e2, S2, ea0, el_W_e[2], el_b[2], W1a, w1b, mlp_b1,
                       prelu_a, mlp_W2, mlp_b2)
    return logits


# speedup vs baseline: 3.2568x; 1.1048x over previous
import functools

import jax
import jax.numpy as jnp
from jax import lax
from jax.experimental import pallas as pl
from jax.experimental.pallas import tpu as pltpu
from jax.experimental.pallas import tpu_sc as plsc

N = 10000
E = 320000
U = 128

NC = 2
NS = 16
NW = NC * NS
PER_W = E // NW
T = 80
NT = PER_W // T
TM = 40
NTM = PER_W // TM

_mesh = functools.partial(
    plsc.VectorSubcoreMesh, core_axis_name="c", subcore_axis_name="s")


def _ring(nt, nb, fire, drain_out, finish):
    for u in range(nb - 1):
        fire(u, u % nb)
    n_iter = -(-nt // nb)

    def body(i, _):
        for j in range(nb):
            t = i * nb + j
            u = t + nb - 1
            b_u = (j + nb - 1) % nb

            @pl.when(t < nt)
            def _():
                finish(t, j)

            @pl.when((u >= nb) & (u < nt))
            def _():
                drain_out(u - nb, b_u)

            @pl.when(u < nt)
            def _():
                fire(u, b_u)
        return 0

    lax.fori_loop(0, n_iter, body, 0)
    for k in range(nb):
        drain_out(nt - nb + k, (nt - nb + k) % nb)


def _sc_msg_agg(Pm, sig, src, dst):
    @functools.partial(
        pl.kernel,
        mesh=_mesh(),
        out_type=jax.ShapeDtypeStruct((NC, N, U), jnp.float32),
        scratch_types=[
            pltpu.VMEM((PER_W,), jnp.int32),
            pltpu.VMEM((TM,), jnp.int32), pltpu.VMEM((TM,), jnp.int32),
            pltpu.VMEM((TM,), jnp.int32),
            pltpu.VMEM((TM, U), jnp.float32), pltpu.VMEM((TM, U), jnp.float32),
            pltpu.VMEM((TM, U), jnp.float32),
            pltpu.VMEM((TM, U), jnp.float32), pltpu.VMEM((TM, U), jnp.float32),
            pltpu.VMEM((TM, U), jnp.float32),
            pltpu.VMEM_SHARED((N, U), jnp.float32),
            pltpu.SemaphoreType.DMA, pltpu.SemaphoreType.DMA,
            pltpu.SemaphoreType.DMA, pltpu.SemaphoreType.DMA,
            pltpu.SemaphoreType.DMA, pltpu.SemaphoreType.DMA,
        ],
    )
    def k(pm_h, sig_h, src_h, dst_h, agg_out,
          six, dx0, dx1, dx2, g0, g1, g2, sg0, sg1, sg2, agg,
          si0, si1, si2, so0, so1, so2):
        cid = lax.axis_index("c")
        sid = lax.axis_index("s")
        wid = sid * NC + cid
        base0 = wid * PER_W
        G = (g0, g1, g2)
        DX = (dx0, dx1, dx2)
        SG = (sg0, sg1, sg2)
        SI = (si0, si1, si2)
        SO = (so0, so1, so2)

        pltpu.sync_copy(src_h.at[pl.ds(base0, PER_W)], six)

        @plsc.parallel_loop(0, TM, 1, unroll=4)
        def _(r):
            for cc in range(U // 16):
                g0[r, pl.ds(cc * 16, 16)] = jnp.zeros((16,), jnp.float32)
        for q in range(-(-(N // TM) // NS)):
            ch = sid + q * NS

            @pl.when(ch < N // TM)
            def _():
                pltpu.sync_copy(g0, agg.at[pl.ds(ch * TM, TM)])
        plsc.subcore_barrier()

        def fire(t, b):
            pltpu.async_copy(pm_h.at[six.at[pl.ds(t * TM, TM)]], G[b], SI[b])
            pltpu.async_copy(sig_h.at[pl.ds(base0 + t * TM, TM)], SG[b], SI[b])
            pltpu.async_copy(dst_h.at[pl.ds(base0 + t * TM, TM)], DX[b], SI[b])

        def drain_out(t, b):
            pltpu.make_async_copy(G[b], agg.at[DX[b]], SO[b]).wait()

        def finish(t, b):
            g, sg = G[b], SG[b]
            pltpu.make_async_copy(pm_h.at[six.at[pl.ds(t * TM, TM)]],
                                  g, SI[b]).wait()
            pltpu.make_async_copy(
                sig_h.at[pl.ds(base0 + t * TM, TM)], sg, SI[b]).wait()
            pltpu.make_async_copy(
                dst_h.at[pl.ds(base0 + t * TM, TM)], DX[b], SI[b]).wait()

            @plsc.parallel_loop(0, TM, 1, unroll=4)
            def _(r):
                for cc in range(U // 16):
                    c = cc * 16
                    g[r, pl.ds(c, 16)] = (g[r, pl.ds(c, 16)]
                                          * sg[r, pl.ds(c, 16)])
            pltpu.async_copy(g, agg.at[DX[b]], SO[b], add=True)

        _ring(NTM, 3, fire, drain_out, finish)
        plsc.subcore_barrier()

        for q in range(-(-(N // TM) // NS)):
            ch = sid + q * NS

            @pl.when(ch < N // TM)
            def _():
                pltpu.sync_copy(agg.at[pl.ds(ch * TM, TM)], g0)
                pltpu.sync_copy(g0, agg_out.at[cid, pl.ds(ch * TM, TM)])

    return k(Pm, sig, src, dst)


def _sc_gather_sum(Ps, Pd, src, dst):
    @functools.partial(
        pl.kernel,
        mesh=_mesh(),
        out_type=jax.ShapeDtypeStruct((E, U), jnp.float32),
        scratch_types=[
            pltpu.VMEM((PER_W,), jnp.int32), pltpu.VMEM((PER_W,), jnp.int32),
            pltpu.VMEM((T, U), jnp.float32), pltpu.VMEM((T, U), jnp.float32),
            pltpu.VMEM((T, U), jnp.float32), pltpu.VMEM((T, U), jnp.float32),
            pltpu.VMEM((T, U), jnp.float32), pltpu.VMEM((T, U), jnp.float32),
            pltpu.VMEM((T, U), jnp.float32), pltpu.VMEM((T, U), jnp.float32),
            pltpu.SemaphoreType.DMA, pltpu.SemaphoreType.DMA,
            pltpu.SemaphoreType.DMA, pltpu.SemaphoreType.DMA,
            pltpu.SemaphoreType.DMA, pltpu.SemaphoreType.DMA,
            pltpu.SemaphoreType.DMA, pltpu.SemaphoreType.DMA,
        ],
    )
    def k(ps_h, pd_h, src_h, dst_h, s_out,
          six, dix, g10, g11, g12, g13, g20, g21, g22, g23,
          si0, si1, si2, si3, so0, so1, so2, so3):
        wid = lax.axis_index("s") * NC + lax.axis_index("c")
        base0 = wid * PER_W
        G1 = (g10, g11, g12, g13)
        G2 = (g20, g21, g22, g23)
        SI = (si0, si1, si2, si3)
        SO = (so0, so1, so2, so3)

        pltpu.sync_copy(src_h.at[pl.ds(base0, PER_W)], six)
        pltpu.sync_copy(dst_h.at[pl.ds(base0, PER_W)], dix)

        def fire(t, b):
            pltpu.async_copy(ps_h.at[six.at[pl.ds(t * T, T)]], G1[b], SI[b])
            pltpu.async_copy(pd_h.at[dix.at[pl.ds(t * T, T)]], G2[b], SI[b])

        def drain_out(t, b):
            pltpu.make_async_copy(ps_h.at[pl.ds(0, T)], G2[b], SO[b]).wait()

        def finish(t, b):
            g1, g2 = G1[b], G2[b]
            pltpu.make_async_copy(ps_h.at[six.at[pl.ds(t * T, T)]],
                                  g1, SI[b]).wait()
            pltpu.make_async_copy(pd_h.at[dix.at[pl.ds(t * T, T)]],
                                  g2, SI[b]).wait()

            @plsc.parallel_loop(0, T, 1, unroll=4)
            def _(r):
                for cc in range(U // 16):
                    c = cc * 16
                    g1[r, pl.ds(c, 16)] = (g1[r, pl.ds(c, 16)]
                                           + g2[r, pl.ds(c, 16)])
            pltpu.async_copy(g1, s_out.at[pl.ds(base0 + t * T, T)], SO[b])

        _ring(NT, 4, fire, drain_out, finish)

    return k(Ps, Pd, src, dst)


EB = 4000


def _tc_node_init(pos, W_pos, b_pos, ws, bs):
    n_proj = len(ws)

    def body(pos_r, wp, bp, *rest):
        w_refs = rest[:n_proj]
        b_refs = rest[n_proj:2 * n_proj]
        outs = rest[2 * n_proj:]
        x = jnp.dot(pos_r[...], wp[...],
                    preferred_element_type=jnp.float32) + bp[...]
        outs[0][...] = x
        for wo, bo, oo in zip(w_refs, b_refs, outs[1:]):
            oo[...] = jnp.dot(x, wo[...],
                              preferred_element_type=jnp.float32) + bo[...]

    outs = (jax.ShapeDtypeStruct((N, U), jnp.float32),) + tuple(
        jax.ShapeDtypeStruct((N, w.shape[1]), jnp.float32) for w in ws)
    return pl.pallas_call(body, out_shape=outs)(
        pos, W_pos, b_pos.reshape(1, U), *ws,
        *[b.reshape(1, -1) for b in bs])


def _tc_node_update(x, R, agg0, agg1, ws, bs):
    n_proj = len(ws)

    def body(x_r, r_r, a0, a1, *rest):
        w_refs = rest[:n_proj]
        b_refs = rest[n_proj:2 * n_proj]
        outs = rest[2 * n_proj:]
        x1 = x_r[...] + jax.nn.relu(r_r[...] + a0[...] + a1[...])
        outs[0][...] = x1
        for wo, bo, oo in zip(w_refs, b_refs, outs[1:]):
            oo[...] = jnp.dot(x1, wo[...],
                              preferred_element_type=jnp.float32) + bo[...]

    outs = (jax.ShapeDtypeStruct((N, U), jnp.float32),) + tuple(
        jax.ShapeDtypeStruct((N, w.shape[1]), jnp.float32) for w in ws)
    return pl.pallas_call(body, out_shape=outs)(
        x, R, agg0, agg1, *ws, *[b.reshape(1, -1) for b in bs])


def _tc_edge_init(edge_attr, W_attr, b_attr):
    def body(ea, wa, ba, e_o, sig_o):
        e = jnp.dot(ea[...], wa[...],
                    preferred_element_type=jnp.float32) + ba[...]
        e_o[...] = e.astype(jnp.bfloat16)
        sig_o[...] = jax.nn.sigmoid(e)

    grid = (E // EB,)
    return pl.pallas_call(
        body,
        grid=grid,
        in_specs=[
            pl.BlockSpec((EB, 5), lambda i: (i, 0)),
            pl.BlockSpec((5, U), lambda i: (0, 0)),
            pl.BlockSpec((1, U), lambda i: (0, 0)),
        ],
        out_specs=(pl.BlockSpec((EB, U), lambda i: (i, 0)),) * 2,
        out_shape=(jax.ShapeDtypeStruct((E, U), jnp.bfloat16),
                   jax.ShapeDtypeStruct((E, U), jnp.float32)),
    )(edge_attr, W_attr, b_attr.reshape(1, U))


def _tc_edge_linear(e, S, We, be, want_sig):
    def body(e_r, s_r, w, b, *outs):
        ev = e_r[...].astype(jnp.float32)
        enew = ev + jax.nn.relu(
            s_r[...] + jnp.dot(e_r[...], w[...],
                               preferred_element_type=jnp.float32)
            + b[...])
        outs[0][...] = enew.astype(jnp.bfloat16)
        if want_sig:
            outs[1][...] = jax.nn.sigmoid(enew)

    n_out = 2 if want_sig else 1
    grid = (E // EB,)
    res = pl.pallas_call(
        body,
        grid=grid,
        in_specs=[
            pl.BlockSpec((EB, U), lambda i: (i, 0)),
            pl.BlockSpec((EB, U), lambda i: (i, 0)),
            pl.BlockSpec((U, U), lambda i: (0, 0)),
            pl.BlockSpec((1, U), lambda i: (0, 0)),
        ],
        out_specs=(pl.BlockSpec((EB, U), lambda i: (i, 0)),) * n_out,
        out_shape=(jax.ShapeDtypeStruct((E, U), jnp.bfloat16),
                   jax.ShapeDtypeStruct((E, U), jnp.float32))[:n_out],
    )(e, S, We, be.reshape(1, U))
    return res if want_sig else (res[0], None)


def _tc_final(e, S, ea0, We, be, W1a, w1b, b1, a, W2, b2):
    H = W1a.shape[1]

    def body(e_r, s_r, ea_r, w, b, w1, w1b_r, b1_r, a_r, w2, b2_r, out_r):
        ev = e_r[...].astype(jnp.float32)
        ef = ev + jax.nn.relu(
            s_r[...] + jnp.dot(e_r[...], w[...],
                               preferred_element_type=jnp.float32)
            + b[...])
        h = (jnp.dot(ef, w1[...], preferred_element_type=jnp.float32)
             + ea_r[...] * w1b_r[...] + b1_r[...])
        h = jnp.where(h >= 0, h, a_r[...] * h)
        out_r[...] = jnp.dot(h, w2[...],
                             preferred_element_type=jnp.float32) + b2_r[...]

    grid = (E // EB,)
    return pl.pallas_call(
        body,
        grid=grid,
        in_specs=[
            pl.BlockSpec((EB, U), lambda i: (i, 0)),
            pl.BlockSpec((EB, U), lambda i: (i, 0)),
            pl.BlockSpec((EB, 1), lambda i: (i, 0)),
            pl.BlockSpec((U, U), lambda i: (0, 0)),
            pl.BlockSpec((1, U), lambda i: (0, 0)),
            pl.BlockSpec((U, H), lambda i: (0, 0)),
            pl.BlockSpec((1, H), lambda i: (0, 0)),
            pl.BlockSpec((1, H), lambda i: (0, 0)),
            pl.BlockSpec((1, 1), lambda i: (0, 0)),
            pl.BlockSpec((H, 1), lambda i: (0, 0)),
            pl.BlockSpec((1, 1), lambda i: (0, 0)),
        ],
        out_specs=pl.BlockSpec((EB, 1), lambda i: (i, 0)),
        out_shape=jax.ShapeDtypeStruct((E, 1), jnp.float32),
    )(e, S, ea0, We, be.reshape(1, U), W1a, w1b, b1.reshape(1, H),
      a.reshape(1, 1), W2, b2.reshape(1, 1))


def kernel(pos, edge_attr, edge_index, W_pos, b_pos, W_attr, b_attr,
           nc_W_root, nc_b_root, nc_W_msg, nc_b_msg, el_W_src, el_W_dst,
           el_W_e, el_b, mlp_W1, mlp_b1, prelu_a, mlp_W2, mlp_b2):
    src = edge_index[0]
    dst = edge_index[1]
    zb = jnp.zeros((U,), jnp.float32)

    x0, Ps0, Pm0, Pd0, R0 = _tc_node_init(
        pos, W_pos, b_pos,
        [el_W_src[0], nc_W_msg[0], el_W_dst[0], nc_W_root[0]],
        [zb, nc_b_msg[0], zb, nc_b_root[0]])
    e0, sig0 = _tc_edge_init(edge_attr, W_attr, b_attr)

    S0 = _sc_gather_sum(Ps0, Pd0, src, dst)
    agg0 = _sc_msg_agg(Pm0, sig0, src, dst)
    e1, sig1 = _tc_edge_linear(e0, S0, el_W_e[0], el_b[0], want_sig=True)
    x1, Ps1, Pm1, Pd1, R1 = _tc_node_update(
        x0, R0, agg0[0], agg0[1],
        [el_W_src[1], nc_W_msg[1], el_W_dst[1], nc_W_root[1]],
        [zb, nc_b_msg[1], zb, nc_b_root[1]])

    S1 = _sc_gather_sum(Ps1, Pd1, src, dst)
    agg1 = _sc_msg_agg(Pm1, sig1, src, dst)
    e2, _ = _tc_edge_linear(e1, S1, el_W_e[1], el_b[1], want_sig=False)
    x2, Ps2, Pd2 = _tc_node_update(
        x1, R1, agg1[0], agg1[1],
        [el_W_src[2], el_W_dst[2]], [zb, zb])

    S2 = _sc_gather_sum(Ps2, Pd2, src, dst)
    ea0 = edge_attr[:, :1]
    W1a = mlp_W1[:U]
    w1b = mlp_W1[U:U + 1]
    logits = _tc_final(e2, S2, ea0, el_W_e[2], el_b[2], W1a, w1b, mlp_b1,
                       prelu_a, mlp_W2, mlp_b2)
    return logits
